# baseline (speedup 1.0000x reference)
import jax
import jax.numpy as jnp
from jax.experimental import pallas as pl

NE = 10000
NN = 10000
D = 128
LABELS = ['same', 'anti', 'ne', 'nn', 'en']


def _mlp(x, layers):
    n = len(layers)
    for i, (W, b) in enumerate(layers):
        x = x @ W + b
        if i < n - 1:
            x = jax.nn.silu(x)
    return x


def kernel(elec_s, elec_v, nuc_s, nuc_v, dist_same, dist_anti, dist_ne, dist_nn, dist_en, dir_same, dir_anti, dir_ne, dir_nn, dir_en, snd_same, snd_anti, snd_ne, snd_nn, snd_en, rcv_same, rcv_anti, rcv_ne, rcv_nn, rcv_en, params):
    dists = {'same': dist_same, 'anti': dist_anti, 'ne': dist_ne, 'nn': dist_nn, 'en': dist_en}
    dirs = {'same': dir_same, 'anti': dir_anti, 'ne': dir_ne, 'nn': dir_nn, 'en': dir_en}
    senders = {'same': snd_same, 'anti': snd_anti, 'ne': snd_ne, 'nn': snd_nn, 'en': snd_en}
    receivers = {'same': rcv_same, 'anti': rcv_anti, 'ne': rcv_ne, 'nn': rcv_nn, 'en': rcv_en}

    h_elec = _mlp(elec_s, params['h'])
    h_nuc = _mlp(nuc_s, params['h'])

    node_map = {
        'ne': (NE, h_nuc, nuc_v),
        'nn': (NN, h_nuc, nuc_v),
        'en': (NN, h_elec, elec_v),
        'same': (NE, h_elec, elec_v),
        'anti': (NE, h_elec, elec_v),
    }
    z = {}
    for lbl in LABELS:
        n_out, h_src, src_v = node_map[lbl]
        we = dists[lbl] @ params['w'][lbl][0][0] + params['w'][lbl][0][1]
        phi = we * h_src[senders[lbl]]
        f_s, f_vv, f_vs = jnp.split(phi, 3, axis=-1)
        z_s = jax.ops.segment_sum(f_s, receivers[lbl], n_out)
        vv = f_vv[..., None] * src_v[senders[lbl]]
        vs = f_vs[..., None] * dirs[lbl][:, None, :]
        z_v = jax.ops.segment_sum(vv + vs, receivers[lbl], n_out)
        z[lbl] = (z_s, z_v)
    upd = {}
    for lbl in LABELS:
        z_s, z_v = z[lbl]
        Vv = jnp.einsum('ndi,de->nei', z_v, params['V'][lbl])
        Uv = jnp.einsum('ndi,de->nei', z_v, params['U'][lbl])
        g = _mlp(jnp.concatenate([z_s, jnp.linalg.norm(Vv, axis=-1)], axis=-1), params['g'][lbl])
        a_ss, a_vv, a_sv = jnp.split(g, 3, axis=-1)
        upd_v = Uv * a_vv[..., None]
        upd_s = a_sv * jnp.einsum('pei,pei->pe', Uv, Vv) + a_ss
        upd[lbl] = (upd_s, upd_v)
    elec_s_new = elec_s + upd['ne'][0] + upd['same'][0] + upd['anti'][0]
    elec_v_new = elec_v + upd['ne'][1] + upd['same'][1] + upd['anti'][1]
    nuc_s_new = nuc_s + upd['nn'][0] + upd['en'][0]
    nuc_v_new = nuc_v + upd['nn'][1] + upd['en'][1]
    return (elec_s_new, elec_v_new, nuc_s_new, nuc_v_new)


# speedup vs baseline: 16.0438x; 16.0438x over previous
import functools

import jax
import jax.numpy as jnp
from jax import lax
from jax.experimental import pallas as pl
from jax.experimental.pallas import tpu as pltpu, tpu_sc as plsc

N = 10000
D = 128
DF = 16
E = 160000
H_PAD = 256
G_PAD = 384
NC, NS = 2, 16
NW = NC * NS
CHUNK = 128
N_CHUNKS = E // CHUNK
G_PER_TILE = -(-N_CHUNKS // NW)
S_PER_CORE = N_CHUNKS // NC
S_PER_TILE = -(-S_PER_CORE // NS)
ZBLK = 624
ZTAIL = N - NS * ZBLK

_mesh = plsc.VectorSubcoreMesh(core_axis_name="c", subcore_axis_name="s")


def _silu(x):
    return x * (1.0 / (1.0 + jnp.exp(-x)))


def _h_mlp_body(x_ref, w1_ref, b1_ref, w2_ref, b2_ref, o_ref):
    h = jnp.dot(x_ref[...], w1_ref[...], preferred_element_type=jnp.float32)
    h = _silu(h + b1_ref[...])
    o_ref[...] = jnp.dot(h, w2_ref[...], preferred_element_type=jnp.float32) + b2_ref[...]


def _h_mlp(x, w1p, b1p, w2p, b2):
    blk = 400
    return pl.pallas_call(
        _h_mlp_body,
        grid=(N // blk,),
        in_specs=[
            pl.BlockSpec((blk, D), lambda i: (i, 0)),
            pl.BlockSpec((D, H_PAD), lambda i: (0, 0)),
            pl.BlockSpec((1, H_PAD), lambda i: (0, 0)),
            pl.BlockSpec((H_PAD, 3 * D), lambda i: (0, 0)),
            pl.BlockSpec((1, 3 * D), lambda i: (0, 0)),
        ],
        out_specs=pl.BlockSpec((blk, 3 * D), lambda i: (i, 0)),
        out_shape=jax.ShapeDtypeStruct((N, 3 * D), jnp.float32),
    )(x, w1p, b1p, w2p, b2)


def _gather_body(h_tab, v0_tab, v1_tab, v2_tab, idx_hbm,
                 rh_out, rv0_out, rv1_out, rv2_out,
                 idx_v, h_buf, v0_buf, v1_buf, v2_buf, sem):
    wid = lax.axis_index("s") * NC + lax.axis_index("c")

    def body(j, carry):
        cid = wid + j * NW

        @pl.when(cid < N_CHUNKS)
        def _():
            base = cid * CHUNK
            pltpu.sync_copy(idx_hbm.at[pl.ds(base, CHUNK)], idx_v)
            pltpu.async_copy(h_tab.at[idx_v], h_buf, sem).wait()
            pltpu.sync_copy(h_buf, rh_out.at[pl.ds(base, CHUNK)])
            pltpu.async_copy(v0_tab.at[idx_v], v0_buf, sem).wait()
            pltpu.sync_copy(v0_buf, rv0_out.at[pl.ds(base, CHUNK)])
            pltpu.async_copy(v1_tab.at[idx_v], v1_buf, sem).wait()
            pltpu.sync_copy(v1_buf, rv1_out.at[pl.ds(base, CHUNK)])
            pltpu.async_copy(v2_tab.at[idx_v], v2_buf, sem).wait()
            pltpu.sync_copy(v2_buf, rv2_out.at[pl.ds(base, CHUNK)])

        return carry

    lax.fori_loop(0, G_PER_TILE, body, 0)


_gather = pl.kernel(
    _gather_body,
    out_type=[
        jax.ShapeDtypeStruct((E, 3 * D), jnp.float32),
        jax.ShapeDtypeStruct((E, D), jnp.float32),
        jax.ShapeDtypeStruct((E, D), jnp.float32),
        jax.ShapeDtypeStruct((E, D), jnp.float32),
    ],
    mesh=_mesh,
    scratch_types=[
        pltpu.VMEM((CHUNK,), jnp.int32),
        pltpu.VMEM((CHUNK, 3 * D), jnp.float32),
        pltpu.VMEM((CHUNK, D), jnp.float32),
        pltpu.VMEM((CHUNK, D), jnp.float32),
        pltpu.VMEM((CHUNK, D), jnp.float32),
        pltpu.SemaphoreType.DMA,
    ],
)


def _msg_body(dist_ref, dir_ref, rh_ref, rv0_ref, rv1_ref, rv2_ref,
              ww_ref, bw_ref, m0, m1, m2, m3):
    we = jnp.dot(dist_ref[...], ww_ref[...], preferred_element_type=jnp.float32)
    phi = (we + bw_ref[...]) * rh_ref[...]
    f_vv = phi[:, D:2 * D]
    f_vs = phi[:, 2 * D:]
    d = dir_ref[...]
    m0[...] = phi[:, :D]
    m1[...] = f_vv * rv0_ref[...] + f_vs * d[:, 0:1]
    m2[...] = f_vv * rv1_ref[...] + f_vs * d[:, 1:2]
    m3[...] = f_vv * rv2_ref[...] + f_vs * d[:, 2:3]


def _msg(dist, dirs, rh, rv0, rv1, rv2, ww, bw):
    blk = 1280
    out = jax.ShapeDtypeStruct((E, D), jnp.float32)
    return pl.pallas_call(
        _msg_body,
        grid=(E // blk,),
        in_specs=[
            pl.BlockSpec((blk, DF), lambda i: (i, 0)),
            pl.BlockSpec((blk, 3), lambda i: (i, 0)),
            pl.BlockSpec((blk, 3 * D), lambda i: (i, 0)),
            pl.BlockSpec((blk, D), lambda i: (i, 0)),
            pl.BlockSpec((blk, D), lambda i: (i, 0)),
            pl.BlockSpec((blk, D), lambda i: (i, 0)),
            pl.BlockSpec((DF, 3 * D), lambda i: (0, 0)),
            pl.BlockSpec((1, 3 * D), lambda i: (0, 0)),
        ],
        out_specs=[pl.BlockSpec((blk, D), lambda i: (i, 0))] * 4,
        out_shape=[out, out, out, out],
    )(dist, dirs, rh, rv0, rv1, rv2, ww, bw)


def _scatter_body(m0, m1, m2, m3, rcv_hbm, zeros_hbm,
                  p0, p1, p2, p3,
                  idx_v, msg_buf, acc, sem):
    c = lax.axis_index("c")
    s = lax.axis_index("s")
    row0 = s * ZBLK

    for msg, out in ((m0, p0), (m1, p1), (m2, p2), (m3, p3)):
        pltpu.sync_copy(zeros_hbm.at[pl.ds(0, ZBLK)], acc.at[pl.ds(row0, ZBLK)])

        @pl.when(s == 0)
        def _():
            pltpu.sync_copy(zeros_hbm.at[pl.ds(0, ZTAIL)],
                            acc.at[pl.ds(NS * ZBLK, ZTAIL)])

        plsc.subcore_barrier()

        def body(j, carry):
            k = s + j * NS

            @pl.when(k < S_PER_CORE)
            def _():
                base = (c * S_PER_CORE + k) * CHUNK
                pltpu.sync_copy(rcv_hbm.at[pl.ds(base, CHUNK)], idx_v)
                pltpu.sync_copy(msg.at[pl.ds(base, CHUNK)], msg_buf)
                pltpu.sync_copy(msg_buf, acc.at[idx_v], add=True)

            return carry

        lax.fori_loop(0, S_PER_TILE, body, 0)
        plsc.subcore_barrier()
        pltpu.sync_copy(acc.at[pl.ds(row0, ZBLK)],
                        out.at[c, pl.ds(row0, ZBLK)])

        @pl.when(s == 0)
        def _():
            pltpu.sync_copy(acc.at[pl.ds(NS * ZBLK, ZTAIL)],
                            out.at[c, pl.ds(NS * ZBLK, ZTAIL)])

        plsc.subcore_barrier()


_scatter = pl.kernel(
    _scatter_body,
    out_type=[jax.ShapeDtypeStruct((NC, N, D), jnp.float32)] * 4,
    mesh=_mesh,
    scratch_types=[
        pltpu.VMEM((CHUNK,), jnp.int32),
        pltpu.VMEM((CHUNK, D), jnp.float32),
        pltpu.VMEM_SHARED((N, D), jnp.float32),
        pltpu.SemaphoreType.DMA,
    ],
)


def _upd_body(s_ref, v_ref, *refs):
    n_lbl = (len(refs) - 4) // 10
    zp = refs[:4 * n_lbl]
    wp = refs[4 * n_lbl:10 * n_lbl]
    os_ref, ov0_ref, ov1_ref, ov2_ref = refs[10 * n_lbl:]

    out_s = s_ref[...]
    out_v = [v_ref[0], v_ref[1], v_ref[2]]
    for l in range(n_lbl):
        zs_p, zv0_p, zv1_p, zv2_p = zp[4 * l:4 * l + 4]
        V_r, U_r, g1_r, gb1_r, g2_r, gb2_r = wp[6 * l:6 * l + 6]
        zs = zs_p[0] + zs_p[1]
        Vm = V_r[...]
        Um = U_r[...]
        Vv = []
        Uv = []
        sq = None
        for zv_p in (zv0_p, zv1_p, zv2_p):
            zv = zv_p[0] + zv_p[1]
            vv = jnp.dot(zv, Vm, preferred_element_type=jnp.float32)
            uv = jnp.dot(zv, Um, preferred_element_type=jnp.float32)
            Vv.append(vv)
            Uv.append(uv)
            sq = vv * vv if sq is None else sq + vv * vv
        norm = jnp.sqrt(sq)
        gin = jnp.concatenate([zs, norm], axis=1)
        g1 = _silu(jnp.dot(gin, g1_r[...], preferred_element_type=jnp.float32)
                   + gb1_r[...])
        g = jnp.dot(g1, g2_r[...], preferred_element_type=jnp.float32) + gb2_r[...]
        a_ss = g[:, :D]
        a_vv = g[:, D:2 * D]
        a_sv = g[:, 2 * D:]
        dot = Uv[0] * Vv[0] + Uv[1] * Vv[1] + Uv[2] * Vv[2]
        out_s = out_s + a_sv * dot + a_ss
        out_v = [out_v[i] + Uv[i] * a_vv for i in range(3)]
    os_ref[...] = out_s
    ov0_ref[...] = out_v[0]
    ov1_ref[...] = out_v[1]
    ov2_ref[...] = out_v[2]


def _upd(s_res, v_planes, z_parts, weights):
    blk = 400
    n_lbl = len(z_parts)
    in_specs = [
        pl.BlockSpec((blk, D), lambda i: (i, 0)),
        pl.BlockSpec((3, blk, D), lambda i: (0, i, 0)),
    ]
    args = [s_res, v_planes]
    for parts in z_parts:
        for p in parts:
            args.append(p)
            in_specs.append(pl.BlockSpec((2, blk, D), lambda i: (0, i, 0)))
    for w6 in weights:
        V_m, U_m, g1, gb1, g2, gb2 = w6
        args += [V_m, U_m, g1, gb1, g2, gb2]
        in_specs += [
            pl.BlockSpec((D, D), lambda i: (0, 0)),
            pl.BlockSpec((D, D), lambda i: (0, 0)),
            pl.BlockSpec((2 * D, G_PAD), lambda i: (0, 0)),
            pl.BlockSpec((1, G_PAD), lambda i: (0, 0)),
            pl.BlockSpec((G_PAD, 3 * D), lambda i: (0, 0)),
            pl.BlockSpec((1, 3 * D), lambda i: (0, 0)),
        ]
    out = jax.ShapeDtypeStruct((N, D), jnp.float32)
    return pl.pallas_call(
        _upd_body,
        grid=(N // blk,),
        in_specs=in_specs,
        out_specs=[pl.BlockSpec((blk, D), lambda i: (i, 0))] * 4,
        out_shape=[out, out, out, out],
    )(*args)


LABELS = ['same', 'anti', 'ne', 'nn', 'en']


def kernel(elec_s, elec_v, nuc_s, nuc_v, dist_same, dist_anti, dist_ne, dist_nn, dist_en, dir_same, dir_anti, dir_ne, dir_nn, dir_en, snd_same, snd_anti, snd_ne, snd_nn, snd_en, rcv_same, rcv_anti, rcv_ne, rcv_nn, rcv_en, params):
    dists = {'same': dist_same, 'anti': dist_anti, 'ne': dist_ne, 'nn': dist_nn, 'en': dist_en}
    dirs = {'same': dir_same, 'anti': dir_anti, 'ne': dir_ne, 'nn': dir_nn, 'en': dir_en}
    snd = {'same': snd_same, 'anti': snd_anti, 'ne': snd_ne, 'nn': snd_nn, 'en': snd_en}
    rcv = {'same': rcv_same, 'anti': rcv_anti, 'ne': rcv_ne, 'nn': rcv_nn, 'en': rcv_en}

    f32 = jnp.float32

    (w1, b1), (w2, b2) = params['h']
    w1p = jnp.pad(w1, ((0, 0), (0, H_PAD - w1.shape[1])))
    b1p = jnp.pad(b1, (0, H_PAD - b1.shape[0])).reshape(1, H_PAD)
    w2p = jnp.pad(w2, ((0, H_PAD - w2.shape[0]), (0, 0)))
    b2p = b2.reshape(1, 3 * D)

    h_elec = _h_mlp(elec_s, w1p, b1p, w2p, b2p)
    h_nuc = _h_mlp(nuc_s, w1p, b1p, w2p, b2p)

    ev = jnp.transpose(elec_v, (2, 0, 1))
    nv = jnp.transpose(nuc_v, (2, 0, 1))

    src_map = {'same': (h_elec, ev), 'anti': (h_elec, ev), 'en': (h_elec, ev),
               'ne': (h_nuc, nv), 'nn': (h_nuc, nv)}

    zeros_blk = jnp.zeros((ZBLK, D), f32)

    z_parts = {}
    for lbl in LABELS:
        h_tab, v_pl = src_map[lbl]
        rh, rv0, rv1, rv2 = _gather(h_tab, v_pl[0], v_pl[1], v_pl[2], snd[lbl])
        ww, bw = params['w'][lbl][0]
        m0, m1, m2, m3 = _msg(dists[lbl], dirs[lbl], rh, rv0, rv1, rv2,
                              ww, bw.reshape(1, 3 * D))
        parts = _scatter(m0, m1, m2, m3, rcv[lbl], zeros_blk)
        z_parts[lbl] = parts

    def upd_weights(lbl):
        (g1, gb1), (g2, gb2) = params['g'][lbl]
        g1p = jnp.pad(g1, ((0, 0), (0, G_PAD - g1.shape[1])))
        gb1p = jnp.pad(gb1, (0, G_PAD - gb1.shape[0])).reshape(1, G_PAD)
        g2p = jnp.pad(g2, ((0, G_PAD - g2.shape[0]), (0, 0)))
        gb2p = gb2.reshape(1, 3 * D)
        return (params['V'][lbl], params['U'][lbl], g1p, gb1p, g2p, gb2p)

    elec_lbls = ['ne', 'same', 'anti']
    nuc_lbls = ['nn', 'en']
    es, ev0, ev1, ev2 = _upd(elec_s, ev, [z_parts[l] for l in elec_lbls],
                             [upd_weights(l) for l in elec_lbls])
    ns_, nv0, nv1, nv2 = _upd(nuc_s, nv, [z_parts[l] for l in nuc_lbls],
                              [upd_weights(l) for l in nuc_lbls])

    elec_v_new = jnp.stack([ev0, ev1, ev2], axis=2)
    nuc_v_new = jnp.stack([nv0, nv1, nv2], axis=2)
    return (es, elec_v_new, ns_, nuc_v_new)


# speedup vs baseline: 19.7286x; 1.2297x over previous
import functools

import jax
import jax.numpy as jnp
from jax import lax
from jax.experimental import pallas as pl
from jax.experimental.pallas import tpu as pltpu, tpu_sc as plsc

N = 10000
D = 128
DF = 16
E = 160000
H_PAD = 256
G_PAD = 384
NC, NS = 2, 16
NW = NC * NS

GCH = 64
G_CHUNKS = E // GCH
G_PER_TILE = -(-G_CHUNKS // NW)
G_PAD_E = NW * G_PER_TILE * GCH

SCH = 128
S_CHUNKS = E // SCH
S_CORE0 = 632
S_PER_TILE = 40
S_IDX_ROWS = 1280
ZBLK = 624
ZTAIL = N - NS * ZBLK

_mesh = plsc.VectorSubcoreMesh(core_axis_name="c", subcore_axis_name="s")


def _silu(x):
    return x * (1.0 / (1.0 + jnp.exp(-x)))


def _h_mlp_body(x_ref, w1_ref, b1_ref, w2_ref, b2_ref, o_ref):
    h = jnp.dot(x_ref[...], w1_ref[...], preferred_element_type=jnp.float32)
    h = _silu(h + b1_ref[...])
    o_ref[...] = jnp.dot(h, w2_ref[...], preferred_element_type=jnp.float32) + b2_ref[...]


def _h_mlp(x, w1p, b1p, w2p, b2):
    blk = 400
    return pl.pallas_call(
        _h_mlp_body,
        grid=(N // blk,),
        in_specs=[
            pl.BlockSpec((blk, D), lambda i: (i, 0)),
            pl.BlockSpec((D, H_PAD), lambda i: (0, 0)),
            pl.BlockSpec((1, H_PAD), lambda i: (0, 0)),
            pl.BlockSpec((H_PAD, 3 * D), lambda i: (0, 0)),
            pl.BlockSpec((1, 3 * D), lambda i: (0, 0)),
        ],
        out_specs=pl.BlockSpec((blk, 3 * D), lambda i: (i, 0)),
        out_shape=jax.ShapeDtypeStruct((N, 3 * D), jnp.float32),
    )(x, w1p, b1p, w2p, b2)


def _gather_body(tab, idx_hbm, rows_out,
                 idx_all, buf0, buf1, gsem0, gsem1, wsem0, wsem1):
    wid = lax.axis_index("s") * NC + lax.axis_index("c")
    pltpu.sync_copy(idx_hbm.at[pl.ds(wid * G_PER_TILE * GCH, G_PER_TILE * GCH)],
                    idx_all)
    bufs = (buf0, buf1)
    gsems = (gsem0, gsem1)
    wsems = (wsem0, wsem1)

    def body(t, carry):
        for b in range(2):
            j = 2 * t + b
            cid = wid * G_PER_TILE + j

            @pl.when((j < G_PER_TILE) & (cid < G_CHUNKS))
            def _():
                @pl.when(j >= 2)
                def _():
                    pltpu.make_async_copy(
                        bufs[b], rows_out.at[pl.ds(0, GCH)], wsems[b]).wait()
                pltpu.async_copy(tab.at[idx_all.at[pl.ds(j * GCH, GCH)]],
                                 bufs[b], gsems[b])

            jm = j - 1
            bm = 1 - b
            cidm = wid * G_PER_TILE + jm

            @pl.when((jm >= 0) & (jm < G_PER_TILE) & (cidm < G_CHUNKS))
            def _():
                pltpu.make_async_copy(
                    tab.at[idx_all.at[pl.ds(0, GCH)]], bufs[bm], gsems[bm]).wait()
                pltpu.async_copy(bufs[bm],
                                 rows_out.at[pl.ds(cidm * GCH, GCH)], wsems[bm])

        return carry

    lax.fori_loop(0, G_PER_TILE // 2 + 1, body, 0)
    pltpu.make_async_copy(buf0, rows_out.at[pl.ds(0, GCH)], wsem0).wait()
    pltpu.make_async_copy(buf1, rows_out.at[pl.ds(0, GCH)], wsem1).wait()


_gather = pl.kernel(
    _gather_body,
    out_type=jax.ShapeDtypeStruct((E, 6 * D), jnp.float32),
    mesh=_mesh,
    scratch_types=[
        pltpu.VMEM((G_PER_TILE * GCH,), jnp.int32),
        pltpu.VMEM((GCH, 6 * D), jnp.float32),
        pltpu.VMEM((GCH, 6 * D), jnp.float32),
        pltpu.SemaphoreType.DMA,
        pltpu.SemaphoreType.DMA,
        pltpu.SemaphoreType.DMA,
        pltpu.SemaphoreType.DMA,
    ],
)


def _msg_body(dist_ref, dir_ref, rows_ref, ww_ref, bw_ref, m0, m1, m2, m3):
    we = jnp.dot(dist_ref[...], ww_ref[...], preferred_element_type=jnp.float32)
    r = rows_ref[...]
    phi = (we + bw_ref[...]) * r[:, :3 * D]
    f_vv = phi[:, D:2 * D]
    f_vs = phi[:, 2 * D:3 * D]
    d = dir_ref[...]
    m0[...] = phi[:, :D]
    m1[...] = f_vv * r[:, 3 * D:4 * D] + f_vs * d[:, 0:1]
    m2[...] = f_vv * r[:, 4 * D:5 * D] + f_vs * d[:, 1:2]
    m3[...] = f_vv * r[:, 5 * D:] + f_vs * d[:, 2:3]


def _msg(dist, dirs, rows, ww, bw):
    blk = 1280
    out = jax.ShapeDtypeStruct((E, D), jnp.float32)
    return pl.pallas_call(
        _msg_body,
        grid=(E // blk,),
        in_specs=[
            pl.BlockSpec((blk, DF), lambda i: (i, 0)),
            pl.BlockSpec((blk, 3), lambda i: (i, 0)),
            pl.BlockSpec((blk, 6 * D), lambda i: (i, 0)),
            pl.BlockSpec((DF, 3 * D), lambda i: (0, 0)),
            pl.BlockSpec((1, 3 * D), lambda i: (0, 0)),
        ],
        out_specs=[pl.BlockSpec((blk, D), lambda i: (i, 0))] * 4,
        out_shape=[out, out, out, out],
    )(dist, dirs, rows, ww, bw)


def _scatter_body(m0, m1, m2, m3, rcv2d, zeros_hbm,
                  p0, p1, p2, p3,
                  idx_all, mbuf0, mbuf1, acc,
                  msem0, msem1, asem0, asem1):
    c = lax.axis_index("c")
    s = lax.axis_index("s")
    row0 = s * ZBLK
    k0 = c * S_CORE0 + s * S_PER_TILE
    limit = S_CORE0 - c * (2 * S_CORE0 - S_CHUNKS)
    pltpu.sync_copy(rcv2d.at[pl.ds(k0, S_PER_TILE)], idx_all)
    mbufs = (mbuf0, mbuf1)
    msems = (msem0, msem1)
    asems = (asem0, asem1)

    for msg, out in ((m0, p0), (m1, p1), (m2, p2), (m3, p3)):
        pltpu.sync_copy(zeros_hbm.at[pl.ds(0, ZBLK)], acc.at[pl.ds(row0, ZBLK)])

        @pl.when(s == 0)
        def _():
            pltpu.sync_copy(zeros_hbm.at[pl.ds(0, ZTAIL)],
                            acc.at[pl.ds(NS * ZBLK, ZTAIL)])

        plsc.subcore_barrier()

        def body(t, carry):
            for b in range(2):
                j = 2 * t + b

                @pl.when(s * S_PER_TILE + j < limit)
                def _():
                    @pl.when(j >= 2)
                    def _():
                        pltpu.make_async_copy(
                            mbufs[b], acc.at[idx_all.at[0]], asems[b]).wait()
                    base = (k0 + j) * SCH
                    pltpu.async_copy(msg.at[pl.ds(base, SCH)],
                                     mbufs[b], msems[b])
                    pltpu.make_async_copy(msg.at[pl.ds(base, SCH)],
                                          mbufs[b], msems[b]).wait()
                    pltpu.async_copy(mbufs[b], acc.at[idx_all.at[j]],
                                     asems[b], add=True)

            return carry

        lax.fori_loop(0, S_PER_TILE // 2, body, 0)
        pltpu.make_async_copy(mbuf0, acc.at[idx_all.at[0]], asem0).wait()
        pltpu.make_async_copy(mbuf1, acc.at[idx_all.at[0]], asem1).wait()
        plsc.subcore_barrier()
        pltpu.sync_copy(acc.at[pl.ds(row0, ZBLK)],
                        out.at[c, pl.ds(row0, ZBLK)])

        @pl.when(s == 0)
        def _():
            pltpu.sync_copy(acc.at[pl.ds(NS * ZBLK, ZTAIL)],
                            out.at[c, pl.ds(NS * ZBLK, ZTAIL)])

        plsc.subcore_barrier()


_scatter = pl.kernel(
    _scatter_body,
    out_type=[jax.ShapeDtypeStruct((NC, N, D), jnp.float32)] * 4,
    mesh=_mesh,
    scratch_types=[
        pltpu.VMEM((S_PER_TILE, SCH), jnp.int32),
        pltpu.VMEM((SCH, D), jnp.float32),
        pltpu.VMEM((SCH, D), jnp.float32),
        pltpu.VMEM_SHARED((N, D), jnp.float32),
        pltpu.SemaphoreType.DMA,
        pltpu.SemaphoreType.DMA,
        pltpu.SemaphoreType.DMA,
        pltpu.SemaphoreType.DMA,
    ],
)


def _upd_body(s_ref, v_ref, *refs):
    n_lbl = (len(refs) - 4) // 10
    zp = refs[:4 * n_lbl]
    wp = refs[4 * n_lbl:10 * n_lbl]
    os_ref, ov0_ref, ov1_ref, ov2_ref = refs[10 * n_lbl:]

    out_s = s_ref[...]
    out_v = [v_ref[0], v_ref[1], v_ref[2]]
    for l in range(n_lbl):
        zs_p, zv0_p, zv1_p, zv2_p = zp[4 * l:4 * l + 4]
        V_r, U_r, g1_r, gb1_r, g2_r, gb2_r = wp[6 * l:6 * l + 6]
        zs = zs_p[0] + zs_p[1]
        Vm = V_r[...]
        Um = U_r[...]
        Vv = []
        Uv = []
        sq = None
        for zv_p in (zv0_p, zv1_p, zv2_p):
            zv = zv_p[0] + zv_p[1]
            vv = jnp.dot(zv, Vm, preferred_element_type=jnp.float32)
            uv = jnp.dot(zv, Um, preferred_element_type=jnp.float32)
            Vv.append(vv)
            Uv.append(uv)
            sq = vv * vv if sq is None else sq + vv * vv
        norm = jnp.sqrt(sq)
        gin = jnp.concatenate([zs, norm], axis=1)
        g1 = _silu(jnp.dot(gin, g1_r[...], preferred_element_type=jnp.float32)
                   + gb1_r[...])
        g = jnp.dot(g1, g2_r[...], preferred_element_type=jnp.float32) + gb2_r[...]
        a_ss = g[:, :D]
        a_vv = g[:, D:2 * D]
        a_sv = g[:, 2 * D:]
        dot = Uv[0] * Vv[0] + Uv[1] * Vv[1] + Uv[2] * Vv[2]
        out_s = out_s + a_sv * dot + a_ss
        out_v = [out_v[i] + Uv[i] * a_vv for i in range(3)]
    os_ref[...] = out_s
    ov0_ref[...] = out_v[0]
    ov1_ref[...] = out_v[1]
    ov2_ref[...] = out_v[2]


def _upd(s_res, v_planes, z_parts, weights):
    blk = 400
    n_lbl = len(z_parts)
    in_specs = [
        pl.BlockSpec((blk, D), lambda i: (i, 0)),
        pl.BlockSpec((3, blk, D), lambda i: (0, i, 0)),
    ]
    args = [s_res, v_planes]
    for parts in z_parts:
        for p in parts:
            args.append(p)
            in_specs.append(pl.BlockSpec((2, blk, D), lambda i: (0, i, 0)))
    for w6 in weights:
        V_m, U_m, g1, gb1, g2, gb2 = w6
        args += [V_m, U_m, g1, gb1, g2, gb2]
        in_specs += [
            pl.BlockSpec((D, D), lambda i: (0, 0)),
            pl.BlockSpec((D, D), lambda i: (0, 0)),
            pl.BlockSpec((2 * D, G_PAD), lambda i: (0, 0)),
            pl.BlockSpec((1, G_PAD), lambda i: (0, 0)),
            pl.BlockSpec((G_PAD, 3 * D), lambda i: (0, 0)),
            pl.BlockSpec((1, 3 * D), lambda i: (0, 0)),
        ]
    out = jax.ShapeDtypeStruct((N, D), jnp.float32)
    return pl.pallas_call(
        _upd_body,
        grid=(N // blk,),
        in_specs=in_specs,
        out_specs=[pl.BlockSpec((blk, D), lambda i: (i, 0))] * 4,
        out_shape=[out, out, out, out],
    )(*args)


LABELS = ['same', 'anti', 'ne', 'nn', 'en']


def kernel(elec_s, elec_v, nuc_s, nuc_v, dist_same, dist_anti, dist_ne, dist_nn, dist_en, dir_same, dir_anti, dir_ne, dir_nn, dir_en, snd_same, snd_anti, snd_ne, snd_nn, snd_en, rcv_same, rcv_anti, rcv_ne, rcv_nn, rcv_en, params):
    dists = {'same': dist_same, 'anti': dist_anti, 'ne': dist_ne, 'nn': dist_nn, 'en': dist_en}
    dirs = {'same': dir_same, 'anti': dir_anti, 'ne': dir_ne, 'nn': dir_nn, 'en': dir_en}
    snd = {'same': snd_same, 'anti': snd_anti, 'ne': snd_ne, 'nn': snd_nn, 'en': snd_en}
    rcv = {'same': rcv_same, 'anti': rcv_anti, 'ne': rcv_ne, 'nn': rcv_nn, 'en': rcv_en}

    f32 = jnp.float32

    (w1, b1), (w2, b2) = params['h']
    w1p = jnp.pad(w1, ((0, 0), (0, H_PAD - w1.shape[1])))
    b1p = jnp.pad(b1, (0, H_PAD - b1.shape[0])).reshape(1, H_PAD)
    w2p = jnp.pad(w2, ((0, H_PAD - w2.shape[0]), (0, 0)))
    b2p = b2.reshape(1, 3 * D)

    h_elec = _h_mlp(elec_s, w1p, b1p, w2p, b2p)
    h_nuc = _h_mlp(nuc_s, w1p, b1p, w2p, b2p)

    ev = jnp.transpose(elec_v, (2, 0, 1))
    nv = jnp.transpose(nuc_v, (2, 0, 1))
    cat_e = jnp.concatenate(
        [h_elec, jnp.transpose(elec_v, (0, 2, 1)).reshape(N, 3 * D)], axis=1)
    cat_n = jnp.concatenate(
        [h_nuc, jnp.transpose(nuc_v, (0, 2, 1)).reshape(N, 3 * D)], axis=1)

    src_map = {'same': cat_e, 'anti': cat_e, 'en': cat_e,
               'ne': cat_n, 'nn': cat_n}

    zeros_blk = jnp.zeros((ZBLK, D), f32)

    z_parts = {}
    for lbl in LABELS:
        snd_p = jnp.pad(snd[lbl], (0, G_PAD_E - E))
        rcv_p = jnp.pad(rcv[lbl], (0, S_IDX_ROWS * SCH - E)).reshape(
            S_IDX_ROWS, SCH)
        rows = _gather(src_map[lbl], snd_p)
        ww, bw = params['w'][lbl][0]
        m0, m1, m2, m3 = _msg(dists[lbl], dirs[lbl], rows,
                              ww, bw.reshape(1, 3 * D))
        parts = _scatter(m0, m1, m2, m3, rcv_p, zeros_blk)
        z_parts[lbl] = parts

    def upd_weights(lbl):
        (g1, gb1), (g2, gb2) = params['g'][lbl]
        g1p = jnp.pad(g1, ((0, 0), (0, G_PAD - g1.shape[1])))
        gb1p = jnp.pad(gb1, (0, G_PAD - gb1.shape[0])).reshape(1, G_PAD)
        g2p = jnp.pad(g2, ((0, G_PAD - g2.shape[0]), (0, 0)))
        gb2p = gb2.reshape(1, 3 * D)
        return (params['V'][lbl], params['U'][lbl], g1p, gb1p, g2p, gb2p)

    elec_lbls = ['ne', 'same', 'anti']
    nuc_lbls = ['nn', 'en']
    es, ev0, ev1, ev2 = _upd(elec_s, ev, [z_parts[l] for l in elec_lbls],
                             [upd_weights(l) for l in elec_lbls])
    ns_, nv0, nv1, nv2 = _upd(nuc_s, nv, [z_parts[l] for l in nuc_lbls],
                              [upd_weights(l) for l in nuc_lbls])

    elec_v_new = jnp.stack([ev0, ev1, ev2], axis=2)
    nuc_v_new = jnp.stack([nv0, nv1, nv2], axis=2)
    return (es, elec_v_new, ns_, nuc_v_new)


# speedup vs baseline: 26.5011x; 1.3433x over previous
import functools

import jax
import jax.numpy as jnp
from jax import lax
from jax.experimental import pallas as pl
from jax.experimental.pallas import tpu as pltpu, tpu_sc as plsc

N = 10000
D = 128
DF = 16
E = 160000
H_PAD = 256
G_PAD = 384
NC, NS = 2, 16
NW = NC * NS

GCH = 128
G_CHUNKS = E // GCH
G_PER_TILE = -(-G_CHUNKS // NW)
G_PAD_E = NW * G_PER_TILE * GCH

SCH = 128
S_CHUNKS = E // SCH
S_CORE0 = 632
S_PER_TILE = 40
S_IDX_ROWS = 1280
ZBLK = 624
ZTAIL = N - NS * ZBLK

_mesh = plsc.VectorSubcoreMesh(core_axis_name="c", subcore_axis_name="s")


def _silu(x):
    return x * (1.0 / (1.0 + jnp.exp(-x)))


def _h_mlp_body(x_ref, w1_ref, b1_ref, w2_ref, b2_ref, o_ref):
    h = jnp.dot(x_ref[...], w1_ref[...], preferred_element_type=jnp.float32)
    h = _silu(h + b1_ref[...])
    o_ref[...] = jnp.dot(h, w2_ref[...], preferred_element_type=jnp.float32) + b2_ref[...]


def _h_mlp(x, w1p, b1p, w2p, b2):
    blk = 400
    return pl.pallas_call(
        _h_mlp_body,
        grid=(N // blk,),
        in_specs=[
            pl.BlockSpec((blk, D), lambda i: (i, 0)),
            pl.BlockSpec((D, H_PAD), lambda i: (0, 0)),
            pl.BlockSpec((1, H_PAD), lambda i: (0, 0)),
            pl.BlockSpec((H_PAD, 3 * D), lambda i: (0, 0)),
            pl.BlockSpec((1, 3 * D), lambda i: (0, 0)),
        ],
        out_specs=pl.BlockSpec((blk, 3 * D), lambda i: (i, 0)),
        out_shape=jax.ShapeDtypeStruct((N, 3 * D), jnp.float32),
    )(x, w1p, b1p, w2p, b2)


def _gather_body(tab, idx_hbm, rows_out,
                 idx_all, buf0, buf1, gsem0, gsem1, wsem0, wsem1):
    wid = lax.axis_index("s") * NC + lax.axis_index("c")
    pltpu.sync_copy(idx_hbm.at[pl.ds(wid * G_PER_TILE * GCH, G_PER_TILE * GCH)],
                    idx_all)
    bufs = (buf0, buf1)
    gsems = (gsem0, gsem1)
    wsems = (wsem0, wsem1)

    def body(t, carry):
        for b in range(2):
            j = 2 * t + b
            cid = wid * G_PER_TILE + j

            @pl.when((j < G_PER_TILE) & (cid < G_CHUNKS))
            def _():
                @pl.when(j >= 2)
                def _():
                    pltpu.make_async_copy(
                        bufs[b], rows_out.at[pl.ds(0, GCH)], wsems[b]).wait()
                pltpu.async_copy(tab.at[idx_all.at[pl.ds(j * GCH, GCH)]],
                                 bufs[b], gsems[b])

            jm = j - 1
            bm = 1 - b
            cidm = wid * G_PER_TILE + jm

            @pl.when((jm >= 0) & (jm < G_PER_TILE) & (cidm < G_CHUNKS))
            def _():
                pltpu.make_async_copy(
                    tab.at[idx_all.at[pl.ds(0, GCH)]], bufs[bm], gsems[bm]).wait()
                pltpu.async_copy(bufs[bm],
                                 rows_out.at[pl.ds(cidm * GCH, GCH)], wsems[bm])

        return carry

    lax.fori_loop(0, G_PER_TILE // 2 + 1, body, 0)
    pltpu.make_async_copy(buf0, rows_out.at[pl.ds(0, GCH)], wsem0).wait()
    pltpu.make_async_copy(buf1, rows_out.at[pl.ds(0, GCH)], wsem1).wait()


_gather = pl.kernel(
    _gather_body,
    out_type=jax.ShapeDtypeStruct((E, 3 * D), jnp.int32),
    mesh=_mesh,
    scratch_types=[
        pltpu.VMEM((G_PER_TILE * GCH,), jnp.int32),
        pltpu.VMEM((GCH, 3 * D), jnp.int32),
        pltpu.VMEM((GCH, 3 * D), jnp.int32),
        pltpu.SemaphoreType.DMA,
        pltpu.SemaphoreType.DMA,
        pltpu.SemaphoreType.DMA,
        pltpu.SemaphoreType.DMA,
    ],
)


def _msg_body(dist_ref, dir_ref, rows_ref, ww_ref, bw_ref, m0, m1, m2, m3):
    we = jnp.dot(dist_ref[...], ww_ref[...], preferred_element_type=jnp.float32)
    r = rows_ref[...]
    h = lax.bitcast_convert_type(jnp.left_shift(r, 16), jnp.float32)
    v = lax.bitcast_convert_type(
        jnp.bitwise_and(r, jnp.int32(-65536)), jnp.float32)
    phi = (we + bw_ref[...]) * h
    f_vv = phi[:, D:2 * D]
    f_vs = phi[:, 2 * D:3 * D]
    d = dir_ref[...]
    m0[...] = phi[:, :D]
    m1[...] = f_vv * v[:, :D] + f_vs * d[:, 0:1]
    m2[...] = f_vv * v[:, D:2 * D] + f_vs * d[:, 1:2]
    m3[...] = f_vv * v[:, 2 * D:] + f_vs * d[:, 2:3]


def _msg(dist, dirs, rows, ww, bw):
    blk = 1280
    out = jax.ShapeDtypeStruct((E, D), jnp.float32)
    return pl.pallas_call(
        _msg_body,
        grid=(E // blk,),
        in_specs=[
            pl.BlockSpec((blk, DF), lambda i: (i, 0)),
            pl.BlockSpec((blk, 3), lambda i: (i, 0)),
            pl.BlockSpec((blk, 3 * D), lambda i: (i, 0)),
            pl.BlockSpec((DF, 3 * D), lambda i: (0, 0)),
            pl.BlockSpec((1, 3 * D), lambda i: (0, 0)),
        ],
        out_specs=[pl.BlockSpec((blk, D), lambda i: (i, 0))] * 4,
        out_shape=[out, out, out, out],
    )(dist, dirs, rows, ww, bw)


def _scatter_body(m0, m1, m2, m3, rcv2d, zeros_hbm,
                  p0, p1, p2, p3,
                  idx_all, mbuf0, mbuf1, acc,
                  msem0, msem1, asem0, asem1):
    c = lax.axis_index("c")
    s = lax.axis_index("s")
    row0 = s * ZBLK
    k0 = c * S_CORE0 + s * S_PER_TILE
    limit = S_CORE0 - c * (2 * S_CORE0 - S_CHUNKS)
    pltpu.sync_copy(rcv2d.at[pl.ds(k0, S_PER_TILE)], idx_all)
    mbufs = (mbuf0, mbuf1)
    msems = (msem0, msem1)
    asems = (asem0, asem1)

    for msg, out in ((m0, p0), (m1, p1), (m2, p2), (m3, p3)):
        pltpu.sync_copy(zeros_hbm.at[pl.ds(0, ZBLK)], acc.at[pl.ds(row0, ZBLK)])

        @pl.when(s == 0)
        def _():
            pltpu.sync_copy(zeros_hbm.at[pl.ds(0, ZTAIL)],
                            acc.at[pl.ds(NS * ZBLK, ZTAIL)])

        plsc.subcore_barrier()

        def body(t, carry):
            for b in range(2):
                j = 2 * t + b

                @pl.when(s * S_PER_TILE + j < limit)
                def _():
                    @pl.when(j >= 2)
                    def _():
                        pltpu.make_async_copy(
                            mbufs[b], acc.at[idx_all.at[0]], asems[b]).wait()
                    base = (k0 + j) * SCH
                    pltpu.async_copy(msg.at[pl.ds(base, SCH)],
                                     mbufs[b], msems[b])
                    pltpu.make_async_copy(msg.at[pl.ds(base, SCH)],
                                          mbufs[b], msems[b]).wait()
                    pltpu.async_copy(mbufs[b], acc.at[idx_all.at[j]],
                                     asems[b], add=True)

            return carry

        lax.fori_loop(0, S_PER_TILE // 2, body, 0)
        pltpu.make_async_copy(mbuf0, acc.at[idx_all.at[0]], asem0).wait()
        pltpu.make_async_copy(mbuf1, acc.at[idx_all.at[0]], asem1).wait()
        plsc.subcore_barrier()
        pltpu.sync_copy(acc.at[pl.ds(row0, ZBLK)],
                        out.at[c, pl.ds(row0, ZBLK)])

        @pl.when(s == 0)
        def _():
            pltpu.sync_copy(acc.at[pl.ds(NS * ZBLK, ZTAIL)],
                            out.at[c, pl.ds(NS * ZBLK, ZTAIL)])

        plsc.subcore_barrier()


_scatter = pl.kernel(
    _scatter_body,
    out_type=[jax.ShapeDtypeStruct((NC, N, D), jnp.float32)] * 4,
    mesh=_mesh,
    scratch_types=[
        pltpu.VMEM((S_PER_TILE, SCH), jnp.int32),
        pltpu.VMEM((SCH, D), jnp.float32),
        pltpu.VMEM((SCH, D), jnp.float32),
        pltpu.VMEM_SHARED((N, D), jnp.float32),
        pltpu.SemaphoreType.DMA,
        pltpu.SemaphoreType.DMA,
        pltpu.SemaphoreType.DMA,
        pltpu.SemaphoreType.DMA,
    ],
)


def _upd_body(s_ref, v_ref, *refs):
    n_lbl = (len(refs) - 4) // 10
    zp = refs[:4 * n_lbl]
    wp = refs[4 * n_lbl:10 * n_lbl]
    os_ref, ov0_ref, ov1_ref, ov2_ref = refs[10 * n_lbl:]

    out_s = s_ref[...]
    out_v = [v_ref[0], v_ref[1], v_ref[2]]
    for l in range(n_lbl):
        zs_p, zv0_p, zv1_p, zv2_p = zp[4 * l:4 * l + 4]
        V_r, U_r, g1_r, gb1_r, g2_r, gb2_r = wp[6 * l:6 * l + 6]
        zs = zs_p[0] + zs_p[1]
        Vm = V_r[...]
        Um = U_r[...]
        Vv = []
        Uv = []
        sq = None
        for zv_p in (zv0_p, zv1_p, zv2_p):
            zv = zv_p[0] + zv_p[1]
            vv = jnp.dot(zv, Vm, preferred_element_type=jnp.float32)
            uv = jnp.dot(zv, Um, preferred_element_type=jnp.float32)
            Vv.append(vv)
            Uv.append(uv)
            sq = vv * vv if sq is None else sq + vv * vv
        norm = jnp.sqrt(sq)
        gin = jnp.concatenate([zs, norm], axis=1)
        g1 = _silu(jnp.dot(gin, g1_r[...], preferred_element_type=jnp.float32)
                   + gb1_r[...])
        g = jnp.dot(g1, g2_r[...], preferred_element_type=jnp.float32) + gb2_r[...]
        a_ss = g[:, :D]
        a_vv = g[:, D:2 * D]
        a_sv = g[:, 2 * D:]
        dot = Uv[0] * Vv[0] + Uv[1] * Vv[1] + Uv[2] * Vv[2]
        out_s = out_s + a_sv * dot + a_ss
        out_v = [out_v[i] + Uv[i] * a_vv for i in range(3)]
    os_ref[...] = out_s
    ov0_ref[...] = out_v[0]
    ov1_ref[...] = out_v[1]
    ov2_ref[...] = out_v[2]


def _upd(s_res, v_planes, z_parts, weights):
    blk = 400
    n_lbl = len(z_parts)
    in_specs = [
        pl.BlockSpec((blk, D), lambda i: (i, 0)),
        pl.BlockSpec((3, blk, D), lambda i: (0, i, 0)),
    ]
    args = [s_res, v_planes]
    for parts in z_parts:
        for p in parts:
            args.append(p)
            in_specs.append(pl.BlockSpec((2, blk, D), lambda i: (0, i, 0)))
    for w6 in weights:
        V_m, U_m, g1, gb1, g2, gb2 = w6
        args += [V_m, U_m, g1, gb1, g2, gb2]
        in_specs += [
            pl.BlockSpec((D, D), lambda i: (0, 0)),
            pl.BlockSpec((D, D), lambda i: (0, 0)),
            pl.BlockSpec((2 * D, G_PAD), lambda i: (0, 0)),
            pl.BlockSpec((1, G_PAD), lambda i: (0, 0)),
            pl.BlockSpec((G_PAD, 3 * D), lambda i: (0, 0)),
            pl.BlockSpec((1, 3 * D), lambda i: (0, 0)),
        ]
    out = jax.ShapeDtypeStruct((N, D), jnp.float32)
    return pl.pallas_call(
        _upd_body,
        grid=(N // blk,),
        in_specs=in_specs,
        out_specs=[pl.BlockSpec((blk, D), lambda i: (i, 0))] * 4,
        out_shape=[out, out, out, out],
    )(*args)


LABELS = ['same', 'anti', 'ne', 'nn', 'en']


def kernel(elec_s, elec_v, nuc_s, nuc_v, dist_same, dist_anti, dist_ne, dist_nn, dist_en, dir_same, dir_anti, dir_ne, dir_nn, dir_en, snd_same, snd_anti, snd_ne, snd_nn, snd_en, rcv_same, rcv_anti, rcv_ne, rcv_nn, rcv_en, params):
    dists = {'same': dist_same, 'anti': dist_anti, 'ne': dist_ne, 'nn': dist_nn, 'en': dist_en}
    dirs = {'same': dir_same, 'anti': dir_anti, 'ne': dir_ne, 'nn': dir_nn, 'en': dir_en}
    snd = {'same': snd_same, 'anti': snd_anti, 'ne': snd_ne, 'nn': snd_nn, 'en': snd_en}
    rcv = {'same': rcv_same, 'anti': rcv_anti, 'ne': rcv_ne, 'nn': rcv_nn, 'en': rcv_en}

    f32 = jnp.float32

    (w1, b1), (w2, b2) = params['h']
    w1p = jnp.pad(w1, ((0, 0), (0, H_PAD - w1.shape[1])))
    b1p = jnp.pad(b1, (0, H_PAD - b1.shape[0])).reshape(1, H_PAD)
    w2p = jnp.pad(w2, ((0, H_PAD - w2.shape[0]), (0, 0)))
    b2p = b2.reshape(1, 3 * D)

    h_elec = _h_mlp(elec_s, w1p, b1p, w2p, b2p)
    h_nuc = _h_mlp(nuc_s, w1p, b1p, w2p, b2p)

    ev = jnp.transpose(elec_v, (2, 0, 1))
    nv = jnp.transpose(nuc_v, (2, 0, 1))
    def _pack(h, vflat):
        pair = jnp.stack([h.astype(jnp.bfloat16),
                          vflat.astype(jnp.bfloat16)], axis=-1)
        return lax.bitcast_convert_type(pair, jnp.int32)

    cat_e = _pack(h_elec, jnp.transpose(elec_v, (0, 2, 1)).reshape(N, 3 * D))
    cat_n = _pack(h_nuc, jnp.transpose(nuc_v, (0, 2, 1)).reshape(N, 3 * D))

    src_map = {'same': cat_e, 'anti': cat_e, 'en': cat_e,
               'ne': cat_n, 'nn': cat_n}

    zeros_blk = jnp.zeros((ZBLK, D), f32)

    z_parts = {}
    for lbl in LABELS:
        snd_p = jnp.pad(snd[lbl], (0, G_PAD_E - E))
        rcv_p = jnp.pad(rcv[lbl], (0, S_IDX_ROWS * SCH - E)).reshape(
            S_IDX_ROWS, SCH)
        rows = _gather(src_map[lbl], snd_p)
        ww, bw = params['w'][lbl][0]
        m0, m1, m2, m3 = _msg(dists[lbl], dirs[lbl], rows,
                              ww, bw.reshape(1, 3 * D))
        parts = _scatter(m0, m1, m2, m3, rcv_p, zeros_blk)
        z_parts[lbl] = parts

    def upd_weights(lbl):
        (g1, gb1), (g2, gb2) = params['g'][lbl]
        g1p = jnp.pad(g1, ((0, 0), (0, G_PAD - g1.shape[1])))
        gb1p = jnp.pad(gb1, (0, G_PAD - gb1.shape[0])).reshape(1, G_PAD)
        g2p = jnp.pad(g2, ((0, G_PAD - g2.shape[0]), (0, 0)))
        gb2p = gb2.reshape(1, 3 * D)
        return (params['V'][lbl], params['U'][lbl], g1p, gb1p, g2p, gb2p)

    elec_lbls = ['ne', 'same', 'anti']
    nuc_lbls = ['nn', 'en']
    es, ev0, ev1, ev2 = _upd(elec_s, ev, [z_parts[l] for l in elec_lbls],
                             [upd_weights(l) for l in elec_lbls])
    ns_, nv0, nv1, nv2 = _upd(nuc_s, nv, [z_parts[l] for l in nuc_lbls],
                              [upd_weights(l) for l in nuc_lbls])

    elec_v_new = jnp.stack([ev0, ev1, ev2], axis=2)
    nuc_v_new = jnp.stack([nv0, nv1, nv2], axis=2)
    return (es, elec_v_new, ns_, nuc_v_new)


# speedup vs baseline: 28.0797x; 1.0596x over previous
import functools

import jax
import jax.numpy as jnp
from jax import lax
from jax.experimental import pallas as pl
from jax.experimental.pallas import tpu as pltpu, tpu_sc as plsc

N = 10000
D = 128
DF = 16
E = 160000
H_PAD = 256
G_PAD = 384
NC, NS = 2, 16
NW = NC * NS

GCH = 128
G_CHUNKS = E // GCH
G_PER_TILE = -(-G_CHUNKS // NW)
G_PAD_E = NW * G_PER_TILE * GCH

SCH = 128
S_CHUNKS = E // SCH
S_PER_TILE = 80
S_IDX_ROWS = 1280
ZBLK = 624
ZTAIL = N - NS * ZBLK
ZVB = 48

_mesh = plsc.VectorSubcoreMesh(core_axis_name="c", subcore_axis_name="s")


def _silu(x):
    return x * (1.0 / (1.0 + jnp.exp(-x)))


def _h_mlp_body(x_ref, w1_ref, b1_ref, w2_ref, b2_ref, o_ref):
    h = jnp.dot(x_ref[...], w1_ref[...], preferred_element_type=jnp.float32)
    h = _silu(h + b1_ref[...])
    o_ref[...] = jnp.dot(h, w2_ref[...], preferred_element_type=jnp.float32) + b2_ref[...]


def _h_mlp(x, w1p, b1p, w2p, b2):
    blk = 400
    return pl.pallas_call(
        _h_mlp_body,
        grid=(N // blk,),
        in_specs=[
            pl.BlockSpec((blk, D), lambda i: (i, 0)),
            pl.BlockSpec((D, H_PAD), lambda i: (0, 0)),
            pl.BlockSpec((1, H_PAD), lambda i: (0, 0)),
            pl.BlockSpec((H_PAD, 3 * D), lambda i: (0, 0)),
            pl.BlockSpec((1, 3 * D), lambda i: (0, 0)),
        ],
        out_specs=pl.BlockSpec((blk, 3 * D), lambda i: (i, 0)),
        out_shape=jax.ShapeDtypeStruct((N, 3 * D), jnp.float32),
    )(x, w1p, b1p, w2p, b2)


def _gather_body(tab, idx_hbm, rows_out,
                 idx_all, buf0, buf1, gsem0, gsem1, wsem0, wsem1):
    wid = lax.axis_index("s") * NC + lax.axis_index("c")
    pltpu.sync_copy(idx_hbm.at[pl.ds(wid * G_PER_TILE * GCH, G_PER_TILE * GCH)],
                    idx_all)
    bufs = (buf0, buf1)
    gsems = (gsem0, gsem1)
    wsems = (wsem0, wsem1)

    def body(t, carry):
        for b in range(2):
            j = 2 * t + b
            cid = wid * G_PER_TILE + j

            @pl.when((j < G_PER_TILE) & (cid < G_CHUNKS))
            def _():
                @pl.when(j >= 2)
                def _():
                    pltpu.make_async_copy(
                        bufs[b], rows_out.at[pl.ds(0, GCH)], wsems[b]).wait()
                pltpu.async_copy(tab.at[idx_all.at[pl.ds(j * GCH, GCH)]],
                                 bufs[b], gsems[b])

            jm = j - 1
            bm = 1 - b
            cidm = wid * G_PER_TILE + jm

            @pl.when((jm >= 0) & (jm < G_PER_TILE) & (cidm < G_CHUNKS))
            def _():
                pltpu.make_async_copy(
                    tab.at[idx_all.at[pl.ds(0, GCH)]], bufs[bm], gsems[bm]).wait()
                pltpu.async_copy(bufs[bm],
                                 rows_out.at[pl.ds(cidm * GCH, GCH)], wsems[bm])

        return carry

    lax.fori_loop(0, G_PER_TILE // 2 + 1, body, 0)
    pltpu.make_async_copy(buf0, rows_out.at[pl.ds(0, GCH)], wsem0).wait()
    pltpu.make_async_copy(buf1, rows_out.at[pl.ds(0, GCH)], wsem1).wait()


_gather = pl.kernel(
    _gather_body,
    out_type=jax.ShapeDtypeStruct((E, 3 * D), jnp.int32),
    mesh=_mesh,
    scratch_types=[
        pltpu.VMEM((G_PER_TILE * GCH,), jnp.int32),
        pltpu.VMEM((GCH, 3 * D), jnp.int32),
        pltpu.VMEM((GCH, 3 * D), jnp.int32),
        pltpu.SemaphoreType.DMA,
        pltpu.SemaphoreType.DMA,
        pltpu.SemaphoreType.DMA,
        pltpu.SemaphoreType.DMA,
    ],
)


def _msg_body(dist_ref, dir_ref, rows_ref, ww_ref, bw_ref, m0, m1, m2, m3):
    we = jnp.dot(dist_ref[...], ww_ref[...], preferred_element_type=jnp.float32)
    r = rows_ref[...]
    h = lax.bitcast_convert_type(jnp.left_shift(r, 16), jnp.float32)
    v = lax.bitcast_convert_type(
        jnp.bitwise_and(r, jnp.int32(-65536)), jnp.float32)
    phi = (we + bw_ref[...]) * h
    f_vv = phi[:, D:2 * D]
    f_vs = phi[:, 2 * D:3 * D]
    d = dir_ref[...]
    m0[...] = phi[:, :D]
    m1[...] = f_vv * v[:, :D] + f_vs * d[:, 0:1]
    m2[...] = f_vv * v[:, D:2 * D] + f_vs * d[:, 1:2]
    m3[...] = f_vv * v[:, 2 * D:] + f_vs * d[:, 2:3]


def _msg(dist, dirs, rows, ww, bw):
    blk = 1280
    out = jax.ShapeDtypeStruct((E, D), jnp.float32)
    return pl.pallas_call(
        _msg_body,
        grid=(E // blk,),
        in_specs=[
            pl.BlockSpec((blk, DF), lambda i: (i, 0)),
            pl.BlockSpec((blk, 3), lambda i: (i, 0)),
            pl.BlockSpec((blk, 3 * D), lambda i: (i, 0)),
            pl.BlockSpec((DF, 3 * D), lambda i: (0, 0)),
            pl.BlockSpec((1, 3 * D), lambda i: (0, 0)),
        ],
        out_specs=[pl.BlockSpec((blk, D), lambda i: (i, 0))] * 4,
        out_shape=[out, out, out, out],
    )(dist, dirs, rows, ww, bw)


def _scatter_body(m0, m1, m2, m3, rcv2d, zeros_hbm,
                  p0, p1, p2, p3,
                  idx_all, zero_v, mbuf0, mbuf1, acc,
                  msem0, msem1, asem0, asem1):
    c = lax.axis_index("c")
    s = lax.axis_index("s")
    row0 = s * ZBLK
    k0 = s * S_PER_TILE
    cnt = jnp.minimum(S_PER_TILE, S_CHUNKS - k0)
    pltpu.sync_copy(rcv2d.at[pl.ds(k0, S_PER_TILE)], idx_all)
    pltpu.sync_copy(zeros_hbm, zero_v)
    mbufs = (mbuf0, mbuf1)
    msems = (msem0, msem1)
    asems = (asem0, asem1)

    for q, (msg, out) in enumerate(((m0, p0), (m1, p1), (m2, p2), (m3, p3))):
        @pl.when(c == q % 2)
        def _(msg=msg, out=out):
            for z in range(ZBLK // ZVB):
                pltpu.sync_copy(zero_v, acc.at[pl.ds(row0 + z * ZVB, ZVB)])

            @pl.when(s == 0)
            def _():
                pltpu.sync_copy(zero_v.at[pl.ds(0, ZTAIL)],
                                acc.at[pl.ds(NS * ZBLK, ZTAIL)])

            plsc.subcore_barrier()

            def body(t, carry):
                for b in range(2):
                    j = 2 * t + b

                    @pl.when(j < cnt)
                    def _():
                        @pl.when(j >= 2)
                        def _():
                            pltpu.make_async_copy(
                                mbufs[b], acc.at[idx_all.at[0]],
                                asems[b]).wait()
                        base = (k0 + j) * SCH
                        pltpu.async_copy(msg.at[pl.ds(base, SCH)],
                                         mbufs[b], msems[b])
                        pltpu.make_async_copy(msg.at[pl.ds(base, SCH)],
                                              mbufs[b], msems[b]).wait()
                        pltpu.async_copy(mbufs[b], acc.at[idx_all.at[j]],
                                         asems[b], add=True)

                return carry

            lax.fori_loop(0, S_PER_TILE // 2, body, 0)
            pltpu.make_async_copy(mbuf0, acc.at[idx_all.at[0]], asem0).wait()
            pltpu.make_async_copy(mbuf1, acc.at[idx_all.at[0]], asem1).wait()
            plsc.subcore_barrier()
            pltpu.sync_copy(acc.at[pl.ds(row0, ZBLK)],
                            out.at[pl.ds(row0, ZBLK)])

            @pl.when(s == 0)
            def _():
                pltpu.sync_copy(acc.at[pl.ds(NS * ZBLK, ZTAIL)],
                                out.at[pl.ds(NS * ZBLK, ZTAIL)])

            plsc.subcore_barrier()


_scatter = pl.kernel(
    _scatter_body,
    out_type=[jax.ShapeDtypeStruct((N, D), jnp.float32)] * 4,
    mesh=_mesh,
    scratch_types=[
        pltpu.VMEM((S_PER_TILE, SCH), jnp.int32),
        pltpu.VMEM((ZVB, D), jnp.float32),
        pltpu.VMEM((SCH, D), jnp.float32),
        pltpu.VMEM((SCH, D), jnp.float32),
        pltpu.VMEM_SHARED((N, D), jnp.float32),
        pltpu.SemaphoreType.DMA,
        pltpu.SemaphoreType.DMA,
        pltpu.SemaphoreType.DMA,
        pltpu.SemaphoreType.DMA,
    ],
)


def _upd_body(s_ref, v_ref, *refs):
    n_lbl = (len(refs) - 4) // 10
    zp = refs[:4 * n_lbl]
    wp = refs[4 * n_lbl:10 * n_lbl]
    os_ref, ov0_ref, ov1_ref, ov2_ref = refs[10 * n_lbl:]

    out_s = s_ref[...]
    out_v = [v_ref[0], v_ref[1], v_ref[2]]
    for l in range(n_lbl):
        zs_p, zv0_p, zv1_p, zv2_p = zp[4 * l:4 * l + 4]
        V_r, U_r, g1_r, gb1_r, g2_r, gb2_r = wp[6 * l:6 * l + 6]
        zs = zs_p[...]
        Vm = V_r[...]
        Um = U_r[...]
        Vv = []
        Uv = []
        sq = None
        for zv_p in (zv0_p, zv1_p, zv2_p):
            zv = zv_p[...]
            vv = jnp.dot(zv, Vm, preferred_element_type=jnp.float32)
            uv = jnp.dot(zv, Um, preferred_element_type=jnp.float32)
            Vv.append(vv)
            Uv.append(uv)
            sq = vv * vv if sq is None else sq + vv * vv
        norm = jnp.sqrt(sq)
        gin = jnp.concatenate([zs, norm], axis=1)
        g1 = _silu(jnp.dot(gin, g1_r[...], preferred_element_type=jnp.float32)
                   + gb1_r[...])
        g = jnp.dot(g1, g2_r[...], preferred_element_type=jnp.float32) + gb2_r[...]
        a_ss = g[:, :D]
        a_vv = g[:, D:2 * D]
        a_sv = g[:, 2 * D:]
        dot = Uv[0] * Vv[0] + Uv[1] * Vv[1] + Uv[2] * Vv[2]
        out_s = out_s + a_sv * dot + a_ss
        out_v = [out_v[i] + Uv[i] * a_vv for i in range(3)]
    os_ref[...] = out_s
    ov0_ref[...] = out_v[0]
    ov1_ref[...] = out_v[1]
    ov2_ref[...] = out_v[2]


def _upd(s_res, v_planes, z_parts, weights):
    blk = 400
    n_lbl = len(z_parts)
    in_specs = [
        pl.BlockSpec((blk, D), lambda i: (i, 0)),
        pl.BlockSpec((3, blk, D), lambda i: (0, i, 0)),
    ]
    args = [s_res, v_planes]
    for parts in z_parts:
        for p in parts:
            args.append(p)
            in_specs.append(pl.BlockSpec((blk, D), lambda i: (i, 0)))
    for w6 in weights:
        V_m, U_m, g1, gb1, g2, gb2 = w6
        args += [V_m, U_m, g1, gb1, g2, gb2]
        in_specs += [
            pl.BlockSpec((D, D), lambda i: (0, 0)),
            pl.BlockSpec((D, D), lambda i: (0, 0)),
            pl.BlockSpec((2 * D, G_PAD), lambda i: (0, 0)),
            pl.BlockSpec((1, G_PAD), lambda i: (0, 0)),
            pl.BlockSpec((G_PAD, 3 * D), lambda i: (0, 0)),
            pl.BlockSpec((1, 3 * D), lambda i: (0, 0)),
        ]
    out = jax.ShapeDtypeStruct((N, D), jnp.float32)
    return pl.pallas_call(
        _upd_body,
        grid=(N // blk,),
        in_specs=in_specs,
        out_specs=[pl.BlockSpec((blk, D), lambda i: (i, 0))] * 4,
        out_shape=[out, out, out, out],
    )(*args)


LABELS = ['same', 'anti', 'ne', 'nn', 'en']


def kernel(elec_s, elec_v, nuc_s, nuc_v, dist_same, dist_anti, dist_ne, dist_nn, dist_en, dir_same, dir_anti, dir_ne, dir_nn, dir_en, snd_same, snd_anti, snd_ne, snd_nn, snd_en, rcv_same, rcv_anti, rcv_ne, rcv_nn, rcv_en, params):
    dists = {'same': dist_same, 'anti': dist_anti, 'ne': dist_ne, 'nn': dist_nn, 'en': dist_en}
    dirs = {'same': dir_same, 'anti': dir_anti, 'ne': dir_ne, 'nn': dir_nn, 'en': dir_en}
    snd = {'same': snd_same, 'anti': snd_anti, 'ne': snd_ne, 'nn': snd_nn, 'en': snd_en}
    rcv = {'same': rcv_same, 'anti': rcv_anti, 'ne': rcv_ne, 'nn': rcv_nn, 'en': rcv_en}

    f32 = jnp.float32

    (w1, b1), (w2, b2) = params['h']
    w1p = jnp.pad(w1, ((0, 0), (0, H_PAD - w1.shape[1])))
    b1p = jnp.pad(b1, (0, H_PAD - b1.shape[0])).reshape(1, H_PAD)
    w2p = jnp.pad(w2, ((0, H_PAD - w2.shape[0]), (0, 0)))
    b2p = b2.reshape(1, 3 * D)

    h_elec = _h_mlp(elec_s, w1p, b1p, w2p, b2p)
    h_nuc = _h_mlp(nuc_s, w1p, b1p, w2p, b2p)

    ev = jnp.transpose(elec_v, (2, 0, 1))
    nv = jnp.transpose(nuc_v, (2, 0, 1))
    def _pack(h, vflat):
        pair = jnp.stack([h.astype(jnp.bfloat16),
                          vflat.astype(jnp.bfloat16)], axis=-1)
        return lax.bitcast_convert_type(pair, jnp.int32)

    cat_e = _pack(h_elec, jnp.transpose(elec_v, (0, 2, 1)).reshape(N, 3 * D))
    cat_n = _pack(h_nuc, jnp.transpose(nuc_v, (0, 2, 1)).reshape(N, 3 * D))

    src_map = {'same': cat_e, 'anti': cat_e, 'en': cat_e,
               'ne': cat_n, 'nn': cat_n}

    zeros_blk = jnp.zeros((ZVB, D), f32)

    z_parts = {}
    for lbl in LABELS:
        snd_p = jnp.pad(snd[lbl], (0, G_PAD_E - E))
        rcv_p = jnp.pad(rcv[lbl], (0, S_IDX_ROWS * SCH - E)).reshape(
            S_IDX_ROWS, SCH)
        rows = _gather(src_map[lbl], snd_p)
        ww, bw = params['w'][lbl][0]
        m0, m1, m2, m3 = _msg(dists[lbl], dirs[lbl], rows,
                              ww, bw.reshape(1, 3 * D))
        parts = _scatter(m0, m1, m2, m3, rcv_p, zeros_blk)
        z_parts[lbl] = parts

    def upd_weights(lbl):
        (g1, gb1), (g2, gb2) = params['g'][lbl]
        g1p = jnp.pad(g1, ((0, 0), (0, G_PAD - g1.shape[1])))
        gb1p = jnp.pad(gb1, (0, G_PAD - gb1.shape[0])).reshape(1, G_PAD)
        g2p = jnp.pad(g2, ((0, G_PAD - g2.shape[0]), (0, 0)))
        gb2p = gb2.reshape(1, 3 * D)
        return (params['V'][lbl], params['U'][lbl], g1p, gb1p, g2p, gb2p)

    elec_lbls = ['ne', 'same', 'anti']
    nuc_lbls = ['nn', 'en']
    es, ev0, ev1, ev2 = _upd(elec_s, ev, [z_parts[l] for l in elec_lbls],
                             [upd_weights(l) for l in elec_lbls])
    ns_, nv0, nv1, nv2 = _upd(nuc_s, nv, [z_parts[l] for l in nuc_lbls],
                              [upd_weights(l) for l in nuc_lbls])

    elec_v_new = jnp.stack([ev0, ev1, ev2], axis=2)
    nuc_v_new = jnp.stack([nv0, nv1, nv2], axis=2)
    return (es, elec_v_new, ns_, nuc_v_new)


# speedup vs baseline: 29.5018x; 1.0506x over previous
import functools

import jax
import jax.numpy as jnp
from jax import lax
from jax.experimental import pallas as pl
from jax.experimental.pallas import tpu as pltpu, tpu_sc as plsc

N = 10000
D = 128
DF = 16
E = 160000
H_PAD = 256
G_PAD = 384
NC, NS = 2, 16
NW = NC * NS

GCH = 128
G_CHUNKS = E // GCH
G_PER_TILE = -(-G_CHUNKS // NW)
G_PAD_E = NW * G_PER_TILE * GCH

SCH = 128
S_CHUNKS = E // SCH
S_PER_TILE = 80
S_IDX_ROWS = 1280
ZBLK = 624
ZTAIL = N - NS * ZBLK
ZVB = 48

_mesh = plsc.VectorSubcoreMesh(core_axis_name="c", subcore_axis_name="s")


def _silu(x):
    return x * (1.0 / (1.0 + jnp.exp(-x)))


def _h_mlp_body(x_ref, w1_ref, b1_ref, w2_ref, b2_ref, o_ref):
    h = jnp.dot(x_ref[...], w1_ref[...], preferred_element_type=jnp.float32)
    h = _silu(h + b1_ref[...])
    o_ref[...] = jnp.dot(h, w2_ref[...], preferred_element_type=jnp.float32) + b2_ref[...]


def _h_mlp(x, w1p, b1p, w2p, b2):
    blk = 400
    return pl.pallas_call(
        _h_mlp_body,
        grid=(N // blk,),
        in_specs=[
            pl.BlockSpec((blk, D), lambda i: (i, 0)),
            pl.BlockSpec((D, H_PAD), lambda i: (0, 0)),
            pl.BlockSpec((1, H_PAD), lambda i: (0, 0)),
            pl.BlockSpec((H_PAD, 3 * D), lambda i: (0, 0)),
            pl.BlockSpec((1, 3 * D), lambda i: (0, 0)),
        ],
        out_specs=pl.BlockSpec((blk, 3 * D), lambda i: (i, 0)),
        out_shape=jax.ShapeDtypeStruct((N, 3 * D), jnp.float32),
    )(x, w1p, b1p, w2p, b2)


def _gather_body(tab, idx_hbm, rows_out,
                 idx_all, buf0, buf1, gsem0, gsem1, wsem0, wsem1):
    wid = lax.axis_index("s") * NC + lax.axis_index("c")
    pltpu.sync_copy(idx_hbm.at[pl.ds(wid * G_PER_TILE * GCH, G_PER_TILE * GCH)],
                    idx_all)
    bufs = (buf0, buf1)
    gsems = (gsem0, gsem1)
    wsems = (wsem0, wsem1)

    def body(t, carry):
        for b in range(2):
            j = 2 * t + b
            cid = wid * G_PER_TILE + j

            @pl.when((j < G_PER_TILE) & (cid < G_CHUNKS))
            def _():
                @pl.when(j >= 2)
                def _():
                    pltpu.make_async_copy(
                        bufs[b], rows_out.at[pl.ds(0, GCH)], wsems[b]).wait()
                pltpu.async_copy(tab.at[idx_all.at[pl.ds(j * GCH, GCH)]],
                                 bufs[b], gsems[b])

            jm = j - 1
            bm = 1 - b
            cidm = wid * G_PER_TILE + jm

            @pl.when((jm >= 0) & (jm < G_PER_TILE) & (cidm < G_CHUNKS))
            def _():
                pltpu.make_async_copy(
                    tab.at[idx_all.at[pl.ds(0, GCH)]], bufs[bm], gsems[bm]).wait()
                pltpu.async_copy(bufs[bm],
                                 rows_out.at[pl.ds(cidm * GCH, GCH)], wsems[bm])

        return carry

    lax.fori_loop(0, G_PER_TILE // 2 + 1, body, 0)
    pltpu.make_async_copy(buf0, rows_out.at[pl.ds(0, GCH)], wsem0).wait()
    pltpu.make_async_copy(buf1, rows_out.at[pl.ds(0, GCH)], wsem1).wait()


_gather = pl.kernel(
    _gather_body,
    out_type=jax.ShapeDtypeStruct((E, 3 * D), jnp.int32),
    mesh=_mesh,
    scratch_types=[
        pltpu.VMEM((G_PER_TILE * GCH,), jnp.int32),
        pltpu.VMEM((GCH, 3 * D), jnp.int32),
        pltpu.VMEM((GCH, 3 * D), jnp.int32),
        pltpu.SemaphoreType.DMA,
        pltpu.SemaphoreType.DMA,
        pltpu.SemaphoreType.DMA,
        pltpu.SemaphoreType.DMA,
    ],
)


def _msg_body(dist_ref, dir_ref, rows_ref, ww_ref, bw_ref, m0, m1, m2, m3):
    we = jnp.dot(dist_ref[...], ww_ref[...], preferred_element_type=jnp.float32)
    r = rows_ref[...]
    h = lax.bitcast_convert_type(jnp.left_shift(r, 16), jnp.float32)
    v = lax.bitcast_convert_type(
        jnp.bitwise_and(r, jnp.int32(-65536)), jnp.float32)
    phi = (we + bw_ref[...]) * h
    f_vv = phi[:, D:2 * D]
    f_vs = phi[:, 2 * D:3 * D]
    d = dir_ref[...]
    m0[...] = phi[:, :D]
    m1[...] = f_vv * v[:, :D] + f_vs * d[:, 0:1]
    m2[...] = f_vv * v[:, D:2 * D] + f_vs * d[:, 1:2]
    m3[...] = f_vv * v[:, 2 * D:] + f_vs * d[:, 2:3]


def _msg(dist, dirs, rows, ww, bw):
    blk = 1280
    out = jax.ShapeDtypeStruct((E, D), jnp.float32)
    return pl.pallas_call(
        _msg_body,
        grid=(E // blk,),
        in_specs=[
            pl.BlockSpec((blk, DF), lambda i: (i, 0)),
            pl.BlockSpec((blk, 3), lambda i: (i, 0)),
            pl.BlockSpec((blk, 3 * D), lambda i: (i, 0)),
            pl.BlockSpec((DF, 3 * D), lambda i: (0, 0)),
            pl.BlockSpec((1, 3 * D), lambda i: (0, 0)),
        ],
        out_specs=[pl.BlockSpec((blk, D), lambda i: (i, 0))] * 4,
        out_shape=[out, out, out, out],
    )(dist, dirs, rows, ww, bw)


def _scatter_body(m0, m1, m2, m3, rcv2d, zeros_hbm,
                  p0, p1, p2, p3,
                  idx_all, zero_v, mbuf0, mbuf1, acc,
                  msem0, msem1, asem0, asem1):
    c = lax.axis_index("c")
    s = lax.axis_index("s")
    row0 = s * ZBLK
    k0 = s * S_PER_TILE
    cnt = jnp.minimum(S_PER_TILE, S_CHUNKS - k0)
    pltpu.sync_copy(rcv2d.at[pl.ds(k0, S_PER_TILE)], idx_all)
    pltpu.sync_copy(zeros_hbm, zero_v)
    mbufs = (mbuf0, mbuf1)
    msems = (msem0, msem1)
    asems = (asem0, asem1)

    for q, (msg, out) in enumerate(((m0, p0), (m1, p1), (m2, p2), (m3, p3))):
        @pl.when(c == q % 2)
        def _(msg=msg, out=out):
            for z in range(ZBLK // ZVB):
                pltpu.sync_copy(zero_v, acc.at[pl.ds(row0 + z * ZVB, ZVB)])

            @pl.when(s == 0)
            def _():
                pltpu.sync_copy(zero_v.at[pl.ds(0, ZTAIL)],
                                acc.at[pl.ds(NS * ZBLK, ZTAIL)])

            plsc.subcore_barrier()

            def body(t, carry):
                for b in range(2):
                    j = 2 * t + b

                    @pl.when(j < cnt)
                    def _():
                        @pl.when(j >= 2)
                        def _():
                            pltpu.make_async_copy(
                                mbufs[b], acc.at[idx_all.at[0]],
                                asems[b]).wait()
                        base = (k0 + j) * SCH
                        pltpu.async_copy(msg.at[pl.ds(base, SCH)],
                                         mbufs[b], msems[b])

                    jm = j - 1
                    bm = 1 - b

                    @pl.when((jm >= 0) & (jm < cnt))
                    def _():
                        pltpu.make_async_copy(
                            msg.at[pl.ds(0, SCH)], mbufs[bm],
                            msems[bm]).wait()
                        pltpu.async_copy(mbufs[bm], acc.at[idx_all.at[jm]],
                                         asems[bm], add=True)

                return carry

            lax.fori_loop(0, S_PER_TILE // 2 + 1, body, 0)
            pltpu.make_async_copy(mbuf0, acc.at[idx_all.at[0]], asem0).wait()
            pltpu.make_async_copy(mbuf1, acc.at[idx_all.at[0]], asem1).wait()
            plsc.subcore_barrier()
            pltpu.sync_copy(acc.at[pl.ds(row0, ZBLK)],
                            out.at[pl.ds(row0, ZBLK)])

            @pl.when(s == 0)
            def _():
                pltpu.sync_copy(acc.at[pl.ds(NS * ZBLK, ZTAIL)],
                                out.at[pl.ds(NS * ZBLK, ZTAIL)])

            plsc.subcore_barrier()


_scatter = pl.kernel(
    _scatter_body,
    out_type=[jax.ShapeDtypeStruct((N, D), jnp.float32)] * 4,
    mesh=_mesh,
    scratch_types=[
        pltpu.VMEM((S_PER_TILE, SCH), jnp.int32),
        pltpu.VMEM((ZVB, D), jnp.float32),
        pltpu.VMEM((SCH, D), jnp.float32),
        pltpu.VMEM((SCH, D), jnp.float32),
        pltpu.VMEM_SHARED((N, D), jnp.float32),
        pltpu.SemaphoreType.DMA,
        pltpu.SemaphoreType.DMA,
        pltpu.SemaphoreType.DMA,
        pltpu.SemaphoreType.DMA,
    ],
)


def _upd_body(s_ref, v_ref, *refs):
    n_lbl = (len(refs) - 4) // 10
    zp = refs[:4 * n_lbl]
    wp = refs[4 * n_lbl:10 * n_lbl]
    os_ref, ov0_ref, ov1_ref, ov2_ref = refs[10 * n_lbl:]

    out_s = s_ref[...]
    out_v = [v_ref[0], v_ref[1], v_ref[2]]
    for l in range(n_lbl):
        zs_p, zv0_p, zv1_p, zv2_p = zp[4 * l:4 * l + 4]
        V_r, U_r, g1_r, gb1_r, g2_r, gb2_r = wp[6 * l:6 * l + 6]
        zs = zs_p[...]
        Vm = V_r[...]
        Um = U_r[...]
        Vv = []
        Uv = []
        sq = None
        for zv_p in (zv0_p, zv1_p, zv2_p):
            zv = zv_p[...]
            vv = jnp.dot(zv, Vm, preferred_element_type=jnp.float32)
            uv = jnp.dot(zv, Um, preferred_element_type=jnp.float32)
            Vv.append(vv)
            Uv.append(uv)
            sq = vv * vv if sq is None else sq + vv * vv
        norm = jnp.sqrt(sq)
        gin = jnp.concatenate([zs, norm], axis=1)
        g1 = _silu(jnp.dot(gin, g1_r[...], preferred_element_type=jnp.float32)
                   + gb1_r[...])
        g = jnp.dot(g1, g2_r[...], preferred_element_type=jnp.float32) + gb2_r[...]
        a_ss = g[:, :D]
        a_vv = g[:, D:2 * D]
        a_sv = g[:, 2 * D:]
        dot = Uv[0] * Vv[0] + Uv[1] * Vv[1] + Uv[2] * Vv[2]
        out_s = out_s + a_sv * dot + a_ss
        out_v = [out_v[i] + Uv[i] * a_vv for i in range(3)]
    os_ref[...] = out_s
    ov0_ref[...] = out_v[0]
    ov1_ref[...] = out_v[1]
    ov2_ref[...] = out_v[2]


def _upd(s_res, v_planes, z_parts, weights):
    blk = 400
    n_lbl = len(z_parts)
    in_specs = [
        pl.BlockSpec((blk, D), lambda i: (i, 0)),
        pl.BlockSpec((3, blk, D), lambda i: (0, i, 0)),
    ]
    args = [s_res, v_planes]
    for parts in z_parts:
        for p in parts:
            args.append(p)
            in_specs.append(pl.BlockSpec((blk, D), lambda i: (i, 0)))
    for w6 in weights:
        V_m, U_m, g1, gb1, g2, gb2 = w6
        args += [V_m, U_m, g1, gb1, g2, gb2]
        in_specs += [
            pl.BlockSpec((D, D), lambda i: (0, 0)),
            pl.BlockSpec((D, D), lambda i: (0, 0)),
            pl.BlockSpec((2 * D, G_PAD), lambda i: (0, 0)),
            pl.BlockSpec((1, G_PAD), lambda i: (0, 0)),
            pl.BlockSpec((G_PAD, 3 * D), lambda i: (0, 0)),
            pl.BlockSpec((1, 3 * D), lambda i: (0, 0)),
        ]
    out = jax.ShapeDtypeStruct((N, D), jnp.float32)
    return pl.pallas_call(
        _upd_body,
        grid=(N // blk,),
        in_specs=in_specs,
        out_specs=[pl.BlockSpec((blk, D), lambda i: (i, 0))] * 4,
        out_shape=[out, out, out, out],
    )(*args)


LABELS = ['same', 'anti', 'ne', 'nn', 'en']


def kernel(elec_s, elec_v, nuc_s, nuc_v, dist_same, dist_anti, dist_ne, dist_nn, dist_en, dir_same, dir_anti, dir_ne, dir_nn, dir_en, snd_same, snd_anti, snd_ne, snd_nn, snd_en, rcv_same, rcv_anti, rcv_ne, rcv_nn, rcv_en, params):
    dists = {'same': dist_same, 'anti': dist_anti, 'ne': dist_ne, 'nn': dist_nn, 'en': dist_en}
    dirs = {'same': dir_same, 'anti': dir_anti, 'ne': dir_ne, 'nn': dir_nn, 'en': dir_en}
    snd = {'same': snd_same, 'anti': snd_anti, 'ne': snd_ne, 'nn': snd_nn, 'en': snd_en}
    rcv = {'same': rcv_same, 'anti': rcv_anti, 'ne': rcv_ne, 'nn': rcv_nn, 'en': rcv_en}

    f32 = jnp.float32

    (w1, b1), (w2, b2) = params['h']
    w1p = jnp.pad(w1, ((0, 0), (0, H_PAD - w1.shape[1])))
    b1p = jnp.pad(b1, (0, H_PAD - b1.shape[0])).reshape(1, H_PAD)
    w2p = jnp.pad(w2, ((0, H_PAD - w2.shape[0]), (0, 0)))
    b2p = b2.reshape(1, 3 * D)

    h_elec = _h_mlp(elec_s, w1p, b1p, w2p, b2p)
    h_nuc = _h_mlp(nuc_s, w1p, b1p, w2p, b2p)

    ev = jnp.transpose(elec_v, (2, 0, 1))
    nv = jnp.transpose(nuc_v, (2, 0, 1))
    def _pack(h, vflat):
        pair = jnp.stack([h.astype(jnp.bfloat16),
                          vflat.astype(jnp.bfloat16)], axis=-1)
        return lax.bitcast_convert_type(pair, jnp.int32)

    cat_e = _pack(h_elec, jnp.transpose(elec_v, (0, 2, 1)).reshape(N, 3 * D))
    cat_n = _pack(h_nuc, jnp.transpose(nuc_v, (0, 2, 1)).reshape(N, 3 * D))

    src_map = {'same': cat_e, 'anti': cat_e, 'en': cat_e,
               'ne': cat_n, 'nn': cat_n}

    zeros_blk = jnp.zeros((ZVB, D), f32)

    z_parts = {}
    for lbl in LABELS:
        snd_p = jnp.pad(snd[lbl], (0, G_PAD_E - E))
        rcv_p = jnp.pad(rcv[lbl], (0, S_IDX_ROWS * SCH - E)).reshape(
            S_IDX_ROWS, SCH)
        rows = _gather(src_map[lbl], snd_p)
        ww, bw = params['w'][lbl][0]
        m0, m1, m2, m3 = _msg(dists[lbl], dirs[lbl], rows,
                              ww, bw.reshape(1, 3 * D))
        parts = _scatter(m0, m1, m2, m3, rcv_p, zeros_blk)
        z_parts[lbl] = parts

    def upd_weights(lbl):
        (g1, gb1), (g2, gb2) = params['g'][lbl]
        g1p = jnp.pad(g1, ((0, 0), (0, G_PAD - g1.shape[1])))
        gb1p = jnp.pad(gb1, (0, G_PAD - gb1.shape[0])).reshape(1, G_PAD)
        g2p = jnp.pad(g2, ((0, G_PAD - g2.shape[0]), (0, 0)))
        gb2p = gb2.reshape(1, 3 * D)
        return (params['V'][lbl], params['U'][lbl], g1p, gb1p, g2p, gb2p)

    elec_lbls = ['ne', 'same', 'anti']
    nuc_lbls = ['nn', 'en']
    es, ev0, ev1, ev2 = _upd(elec_s, ev, [z_parts[l] for l in elec_lbls],
                             [upd_weights(l) for l in elec_lbls])
    ns_, nv0, nv1, nv2 = _upd(nuc_s, nv, [z_parts[l] for l in nuc_lbls],
                              [upd_weights(l) for l in nuc_lbls])

    elec_v_new = jnp.stack([ev0, ev1, ev2], axis=2)
    nuc_v_new = jnp.stack([nv0, nv1, nv2], axis=2)
    return (es, elec_v_new, ns_, nuc_v_new)


# speedup vs baseline: 29.5609x; 1.0020x over previous
import functools

import jax
import jax.numpy as jnp
from jax import lax
from jax.experimental import pallas as pl
from jax.experimental.pallas import tpu as pltpu, tpu_sc as plsc

N = 10000
D = 128
DF = 16
E = 160000
H_PAD = 256
G_PAD = 384
NC, NS = 2, 16
NW = NC * NS

GCH = 160
G_CHUNKS = E // GCH
G_PER_TILE = -(-G_CHUNKS // NW)
G_PAD_E = NW * G_PER_TILE * GCH

SCH = 128
S_CHUNKS = E // SCH
S_PER_TILE = 80
S_IDX_ROWS = 1280
ZBLK = 624
ZTAIL = N - NS * ZBLK
ZVB = 48

_mesh = plsc.VectorSubcoreMesh(core_axis_name="c", subcore_axis_name="s")


def _silu(x):
    return x * (1.0 / (1.0 + jnp.exp(-x)))


def _h_mlp_body(x_ref, w1_ref, b1_ref, w2_ref, b2_ref, o_ref):
    h = jnp.dot(x_ref[...], w1_ref[...], preferred_element_type=jnp.float32)
    h = _silu(h + b1_ref[...])
    o_ref[...] = jnp.dot(h, w2_ref[...], preferred_element_type=jnp.float32) + b2_ref[...]


def _h_mlp(x, w1p, b1p, w2p, b2):
    blk = 400
    return pl.pallas_call(
        _h_mlp_body,
        grid=(N // blk,),
        in_specs=[
            pl.BlockSpec((blk, D), lambda i: (i, 0)),
            pl.BlockSpec((D, H_PAD), lambda i: (0, 0)),
            pl.BlockSpec((1, H_PAD), lambda i: (0, 0)),
            pl.BlockSpec((H_PAD, 3 * D), lambda i: (0, 0)),
            pl.BlockSpec((1, 3 * D), lambda i: (0, 0)),
        ],
        out_specs=pl.BlockSpec((blk, 3 * D), lambda i: (i, 0)),
        out_shape=jax.ShapeDtypeStruct((N, 3 * D), jnp.float32),
    )(x, w1p, b1p, w2p, b2)


def _gather_body(tab, idx_hbm, rows_out,
                 idx_all, buf0, buf1, gsem0, gsem1, wsem0, wsem1):
    wid = lax.axis_index("s") * NC + lax.axis_index("c")
    pltpu.sync_copy(idx_hbm.at[pl.ds(wid * G_PER_TILE * GCH, G_PER_TILE * GCH)],
                    idx_all)
    bufs = (buf0, buf1)
    gsems = (gsem0, gsem1)
    wsems = (wsem0, wsem1)

    def body(t, carry):
        for b in range(2):
            j = 2 * t + b
            cid = wid * G_PER_TILE + j

            @pl.when((j < G_PER_TILE) & (cid < G_CHUNKS))
            def _():
                @pl.when(j >= 2)
                def _():
                    pltpu.make_async_copy(
                        bufs[b], rows_out.at[pl.ds(0, GCH)], wsems[b]).wait()
                pltpu.async_copy(tab.at[idx_all.at[pl.ds(j * GCH, GCH)]],
                                 bufs[b], gsems[b])

            jm = j - 1
            bm = 1 - b
            cidm = wid * G_PER_TILE + jm

            @pl.when((jm >= 0) & (jm < G_PER_TILE) & (cidm < G_CHUNKS))
            def _():
                pltpu.make_async_copy(
                    tab.at[idx_all.at[pl.ds(0, GCH)]], bufs[bm], gsems[bm]).wait()
                pltpu.async_copy(bufs[bm],
                                 rows_out.at[pl.ds(cidm * GCH, GCH)], wsems[bm])

        return carry

    lax.fori_loop(0, G_PER_TILE // 2 + 1, body, 0)
    pltpu.make_async_copy(buf0, rows_out.at[pl.ds(0, GCH)], wsem0).wait()
    pltpu.make_async_copy(buf1, rows_out.at[pl.ds(0, GCH)], wsem1).wait()


_gather = pl.kernel(
    _gather_body,
    out_type=jax.ShapeDtypeStruct((E, 3 * D), jnp.int32),
    mesh=_mesh,
    scratch_types=[
        pltpu.VMEM((G_PER_TILE * GCH,), jnp.int32),
        pltpu.VMEM((GCH, 3 * D), jnp.int32),
        pltpu.VMEM((GCH, 3 * D), jnp.int32),
        pltpu.SemaphoreType.DMA,
        pltpu.SemaphoreType.DMA,
        pltpu.SemaphoreType.DMA,
        pltpu.SemaphoreType.DMA,
    ],
)


def _msg_body(dist_ref, dir_ref, rows_ref, ww_ref, bw_ref, m0, m1, m2, m3):
    we = jnp.dot(dist_ref[...], ww_ref[...], preferred_element_type=jnp.float32)
    r = rows_ref[...]
    h = lax.bitcast_convert_type(jnp.left_shift(r, 16), jnp.float32)
    v = lax.bitcast_convert_type(
        jnp.bitwise_and(r, jnp.int32(-65536)), jnp.float32)
    phi = (we + bw_ref[...]) * h
    f_vv = phi[:, D:2 * D]
    f_vs = phi[:, 2 * D:3 * D]
    d = dir_ref[...]
    m0[...] = phi[:, :D]
    m1[...] = f_vv * v[:, :D] + f_vs * d[:, 0:1]
    m2[...] = f_vv * v[:, D:2 * D] + f_vs * d[:, 1:2]
    m3[...] = f_vv * v[:, 2 * D:] + f_vs * d[:, 2:3]


def _msg(dist, dirs, rows, ww, bw):
    blk = 1280
    out = jax.ShapeDtypeStruct((E, D), jnp.float32)
    return pl.pallas_call(
        _msg_body,
        grid=(E // blk,),
        in_specs=[
            pl.BlockSpec((blk, DF), lambda i: (i, 0)),
            pl.BlockSpec((blk, 3), lambda i: (i, 0)),
            pl.BlockSpec((blk, 3 * D), lambda i: (i, 0)),
            pl.BlockSpec((DF, 3 * D), lambda i: (0, 0)),
            pl.BlockSpec((1, 3 * D), lambda i: (0, 0)),
        ],
        out_specs=[pl.BlockSpec((blk, D), lambda i: (i, 0))] * 4,
        out_shape=[out, out, out, out],
    )(dist, dirs, rows, ww, bw)


def _scatter_body(m0, m1, m2, m3, rcv2d, zeros_hbm,
                  p0, p1, p2, p3,
                  idx_all, zero_v, mbuf0, mbuf1, acc,
                  msem0, msem1, asem0, asem1):
    c = lax.axis_index("c")
    s = lax.axis_index("s")
    row0 = s * ZBLK
    k0 = s * S_PER_TILE
    cnt = jnp.minimum(S_PER_TILE, S_CHUNKS - k0)
    pltpu.sync_copy(rcv2d.at[pl.ds(k0, S_PER_TILE)], idx_all)
    pltpu.sync_copy(zeros_hbm, zero_v)
    mbufs = (mbuf0, mbuf1)
    msems = (msem0, msem1)
    asems = (asem0, asem1)

    for q, (msg, out) in enumerate(((m0, p0), (m1, p1), (m2, p2), (m3, p3))):
        @pl.when(c == q % 2)
        def _(msg=msg, out=out):
            for z in range(ZBLK // ZVB):
                pltpu.sync_copy(zero_v, acc.at[pl.ds(row0 + z * ZVB, ZVB)])

            @pl.when(s == 0)
            def _():
                pltpu.sync_copy(zero_v.at[pl.ds(0, ZTAIL)],
                                acc.at[pl.ds(NS * ZBLK, ZTAIL)])

            plsc.subcore_barrier()

            def body(t, carry):
                for b in range(2):
                    j = 2 * t + b

                    @pl.when(j < cnt)
                    def _():
                        @pl.when(j >= 2)
                        def _():
                            pltpu.make_async_copy(
                                mbufs[b], acc.at[idx_all.at[0]],
                                asems[b]).wait()
                        base = (k0 + j) * SCH
                        pltpu.async_copy(msg.at[pl.ds(base, SCH)],
                                         mbufs[b], msems[b])

                    jm = j - 1
                    bm = 1 - b

                    @pl.when((jm >= 0) & (jm < cnt))
                    def _():
                        pltpu.make_async_copy(
                            msg.at[pl.ds(0, SCH)], mbufs[bm],
                            msems[bm]).wait()
                        pltpu.async_copy(mbufs[bm], acc.at[idx_all.at[jm]],
                                         asems[bm], add=True)

                return carry

            lax.fori_loop(0, S_PER_TILE // 2 + 1, body, 0)
            pltpu.make_async_copy(mbuf0, acc.at[idx_all.at[0]], asem0).wait()
            pltpu.make_async_copy(mbuf1, acc.at[idx_all.at[0]], asem1).wait()
            plsc.subcore_barrier()
            pltpu.sync_copy(acc.at[pl.ds(row0, ZBLK)],
                            out.at[pl.ds(row0, ZBLK)])

            @pl.when(s == 0)
            def _():
                pltpu.sync_copy(acc.at[pl.ds(NS * ZBLK, ZTAIL)],
                                out.at[pl.ds(NS * ZBLK, ZTAIL)])

            plsc.subcore_barrier()


_scatter = pl.kernel(
    _scatter_body,
    out_type=[jax.ShapeDtypeStruct((N, D), jnp.float32)] * 4,
    mesh=_mesh,
    scratch_types=[
        pltpu.VMEM((S_PER_TILE, SCH), jnp.int32),
        pltpu.VMEM((ZVB, D), jnp.float32),
        pltpu.VMEM((SCH, D), jnp.float32),
        pltpu.VMEM((SCH, D), jnp.float32),
        pltpu.VMEM_SHARED((N, D), jnp.float32),
        pltpu.SemaphoreType.DMA,
        pltpu.SemaphoreType.DMA,
        pltpu.SemaphoreType.DMA,
        pltpu.SemaphoreType.DMA,
    ],
)


def _upd_body(s_ref, v_ref, *refs):
    n_lbl = (len(refs) - 4) // 10
    zp = refs[:4 * n_lbl]
    wp = refs[4 * n_lbl:10 * n_lbl]
    os_ref, ov0_ref, ov1_ref, ov2_ref = refs[10 * n_lbl:]

    out_s = s_ref[...]
    out_v = [v_ref[0], v_ref[1], v_ref[2]]
    for l in range(n_lbl):
        zs_p, zv0_p, zv1_p, zv2_p = zp[4 * l:4 * l + 4]
        V_r, U_r, g1_r, gb1_r, g2_r, gb2_r = wp[6 * l:6 * l + 6]
        zs = zs_p[...]
        Vm = V_r[...]
        Um = U_r[...]
        Vv = []
        Uv = []
        sq = None
        for zv_p in (zv0_p, zv1_p, zv2_p):
            zv = zv_p[...]
            vv = jnp.dot(zv, Vm, preferred_element_type=jnp.float32)
            uv = jnp.dot(zv, Um, preferred_element_type=jnp.float32)
            Vv.append(vv)
            Uv.append(uv)
            sq = vv * vv if sq is None else sq + vv * vv
        norm = jnp.sqrt(sq)
        gin = jnp.concatenate([zs, norm], axis=1)
        g1 = _silu(jnp.dot(gin, g1_r[...], preferred_element_type=jnp.float32)
                   + gb1_r[...])
        g = jnp.dot(g1, g2_r[...], preferred_element_type=jnp.float32) + gb2_r[...]
        a_ss = g[:, :D]
        a_vv = g[:, D:2 * D]
        a_sv = g[:, 2 * D:]
        dot = Uv[0] * Vv[0] + Uv[1] * Vv[1] + Uv[2] * Vv[2]
        out_s = out_s + a_sv * dot + a_ss
        out_v = [out_v[i] + Uv[i] * a_vv for i in range(3)]
    os_ref[...] = out_s
    ov0_ref[...] = out_v[0]
    ov1_ref[...] = out_v[1]
    ov2_ref[...] = out_v[2]


def _upd(s_res, v_planes, z_parts, weights):
    blk = 400
    n_lbl = len(z_parts)
    in_specs = [
        pl.BlockSpec((blk, D), lambda i: (i, 0)),
        pl.BlockSpec((3, blk, D), lambda i: (0, i, 0)),
    ]
    args = [s_res, v_planes]
    for parts in z_parts:
        for p in parts:
            args.append(p)
            in_specs.append(pl.BlockSpec((blk, D), lambda i: (i, 0)))
    for w6 in weights:
        V_m, U_m, g1, gb1, g2, gb2 = w6
        args += [V_m, U_m, g1, gb1, g2, gb2]
        in_specs += [
            pl.BlockSpec((D, D), lambda i: (0, 0)),
            pl.BlockSpec((D, D), lambda i: (0, 0)),
            pl.BlockSpec((2 * D, G_PAD), lambda i: (0, 0)),
            pl.BlockSpec((1, G_PAD), lambda i: (0, 0)),
            pl.BlockSpec((G_PAD, 3 * D), lambda i: (0, 0)),
            pl.BlockSpec((1, 3 * D), lambda i: (0, 0)),
        ]
    out = jax.ShapeDtypeStruct((N, D), jnp.float32)
    return pl.pallas_call(
        _upd_body,
        grid=(N // blk,),
        in_specs=in_specs,
        out_specs=[pl.BlockSpec((blk, D), lambda i: (i, 0))] * 4,
        out_shape=[out, out, out, out],
    )(*args)


LABELS = ['same', 'anti', 'ne', 'nn', 'en']


def kernel(elec_s, elec_v, nuc_s, nuc_v, dist_same, dist_anti, dist_ne, dist_nn, dist_en, dir_same, dir_anti, dir_ne, dir_nn, dir_en, snd_same, snd_anti, snd_ne, snd_nn, snd_en, rcv_same, rcv_anti, rcv_ne, rcv_nn, rcv_en, params):
    dists = {'same': dist_same, 'anti': dist_anti, 'ne': dist_ne, 'nn': dist_nn, 'en': dist_en}
    dirs = {'same': dir_same, 'anti': dir_anti, 'ne': dir_ne, 'nn': dir_nn, 'en': dir_en}
    snd = {'same': snd_same, 'anti': snd_anti, 'ne': snd_ne, 'nn': snd_nn, 'en': snd_en}
    rcv = {'same': rcv_same, 'anti': rcv_anti, 'ne': rcv_ne, 'nn': rcv_nn, 'en': rcv_en}

    f32 = jnp.float32

    (w1, b1), (w2, b2) = params['h']
    w1p = jnp.pad(w1, ((0, 0), (0, H_PAD - w1.shape[1])))
    b1p = jnp.pad(b1, (0, H_PAD - b1.shape[0])).reshape(1, H_PAD)
    w2p = jnp.pad(w2, ((0, H_PAD - w2.shape[0]), (0, 0)))
    b2p = b2.reshape(1, 3 * D)

    h_elec = _h_mlp(elec_s, w1p, b1p, w2p, b2p)
    h_nuc = _h_mlp(nuc_s, w1p, b1p, w2p, b2p)

    ev = jnp.transpose(elec_v, (2, 0, 1))
    nv = jnp.transpose(nuc_v, (2, 0, 1))
    def _pack(h, vflat):
        pair = jnp.stack([h.astype(jnp.bfloat16),
                          vflat.astype(jnp.bfloat16)], axis=-1)
        return lax.bitcast_convert_type(pair, jnp.int32)

    cat_e = _pack(h_elec, jnp.transpose(elec_v, (0, 2, 1)).reshape(N, 3 * D))
    cat_n = _pack(h_nuc, jnp.transpose(nuc_v, (0, 2, 1)).reshape(N, 3 * D))

    src_map = {'same': cat_e, 'anti': cat_e, 'en': cat_e,
               'ne': cat_n, 'nn': cat_n}

    zeros_blk = jnp.zeros((ZVB, D), f32)

    z_parts = {}
    for lbl in LABELS:
        snd_p = jnp.pad(snd[lbl], (0, G_PAD_E - E))
        rcv_p = jnp.pad(rcv[lbl], (0, S_IDX_ROWS * SCH - E)).reshape(
            S_IDX_ROWS, SCH)
        rows = _gather(src_map[lbl], snd_p)
        ww, bw = params['w'][lbl][0]
        m0, m1, m2, m3 = _msg(dists[lbl], dirs[lbl], rows,
                              ww, bw.reshape(1, 3 * D))
        parts = _scatter(m0, m1, m2, m3, rcv_p, zeros_blk)
        z_parts[lbl] = parts

    def upd_weights(lbl):
        (g1, gb1), (g2, gb2) = params['g'][lbl]
        g1p = jnp.pad(g1, ((0, 0), (0, G_PAD - g1.shape[1])))
        gb1p = jnp.pad(gb1, (0, G_PAD - gb1.shape[0])).reshape(1, G_PAD)
        g2p = jnp.pad(g2, ((0, G_PAD - g2.shape[0]), (0, 0)))
        gb2p = gb2.reshape(1, 3 * D)
        return (params['V'][lbl], params['U'][lbl], g1p, gb1p, g2p, gb2p)

    elec_lbls = ['ne', 'same', 'anti']
    nuc_lbls = ['nn', 'en']
    es, ev0, ev1, ev2 = _upd(elec_s, ev, [z_parts[l] for l in elec_lbls],
                             [upd_weights(l) for l in elec_lbls])
    ns_, nv0, nv1, nv2 = _upd(nuc_s, nv, [z_parts[l] for l in nuc_lbls],
                              [upd_weights(l) for l in nuc_lbls])

    elec_v_new = jnp.stack([ev0, ev1, ev2], axis=2)
    nuc_v_new = jnp.stack([nv0, nv1, nv2], axis=2)
    return (es, elec_v_new, ns_, nuc_v_new)
